# Initial kernel scaffold; baseline (speedup 1.0000x reference)
import functools

import jax
import jax.numpy as jnp
from jax import lax
from jax.experimental import pallas as pl
from jax.experimental.pallas import tpu as pltpu
from jax.experimental.pallas import tpu_sc as plsc

N = 10000
E = 160000
E_PAD = 163840
CHUNK = 128
N_ACC = 10016
NT = 16
RPT = N // NT
BM = 400

_MESH = dict(core_axis_name="c", subcore_axis_name="s")


def _deg_body(dst_hbm, vals_hbm, out_hbm, idx_d, ones_v, acc, sem):
    c = lax.axis_index("c")
    s = lax.axis_index("s")
    pltpu.sync_copy(vals_hbm.at[c], ones_v)
    for k in range(5):
        pltpu.sync_copy(ones_v, acc.at[pl.ds(s * 640 + k * CHUNK, CHUNK)])
    plsc.subcore_barrier()
    ept = E_PAD // (2 * NT)
    base0 = (c * NT + s) * ept

    def body(i, carry):
        pltpu.sync_copy(dst_hbm.at[pl.ds(base0 + i * CHUNK, CHUNK)], idx_d)
        pltpu.sync_copy(ones_v, acc.at[idx_d], add=True)
        return carry

    lax.fori_loop(0, ept // CHUNK, body, 0)
    plsc.subcore_barrier()
    pltpu.sync_copy(acc.at[pl.ds(s * RPT, RPT)],
                    out_hbm.at[pl.ds(c * N + s * RPT, RPT)])


@functools.partial(
    pl.kernel,
    out_type=jax.ShapeDtypeStruct((2 * N, 16), jnp.float32),
    mesh=plsc.VectorSubcoreMesh(**_MESH),
    scratch_types=[
        pltpu.VMEM((CHUNK,), jnp.int32),
        pltpu.VMEM((CHUNK, 16), jnp.float32),
        pltpu.VMEM_SHARED((10240, 16), jnp.float32),
        pltpu.SemaphoreType.DMA,
    ],
)
def _deg(dst_hbm, vals_hbm, out_hbm, idx_d, ones_v, acc, sem):
    _deg_body(dst_hbm, vals_hbm, out_hbm, idx_d, ones_v, acc, sem)


def _make_prop(cs):
    def body(y_hbm, src_hbm, dst_hbm, out_hbm, idx_s, idx_d, rows, acc, sem):
        c = lax.axis_index("c")
        s = lax.axis_index("s")
        coff = c * N
        pltpu.sync_copy(y_hbm.at[pl.ds(coff + s * RPT, RPT)],
                        acc.at[pl.ds(s * RPT, RPT)])
        plsc.subcore_barrier()
        ept = E_PAD // NT
        base0 = s * ept

        def step(i, carry):
            base = base0 + i * CHUNK
            pltpu.sync_copy(src_hbm.at[pl.ds(base, CHUNK)], idx_s)
            pltpu.sync_copy(dst_hbm.at[pl.ds(base, CHUNK)], idx_d)

            def addoff(j, cc):
                idx_s[pl.ds(j * 16, 16)] = idx_s[pl.ds(j * 16, 16)] + coff
                return cc

            lax.fori_loop(0, CHUNK // 16, addoff, 0)
            pltpu.async_copy(y_hbm.at[idx_s], rows, sem).wait()
            pltpu.sync_copy(rows, acc.at[idx_d], add=True)
            return carry

        lax.fori_loop(0, ept // CHUNK, step, 0)
        plsc.subcore_barrier()
        pltpu.sync_copy(acc.at[pl.ds(s * RPT, RPT)],
                        out_hbm.at[pl.ds(coff + s * RPT, RPT)])

    return pl.kernel(
        body,
        out_type=jax.ShapeDtypeStruct((2 * N, cs), jnp.float32),
        mesh=plsc.VectorSubcoreMesh(**_MESH),
        scratch_types=[
            pltpu.VMEM((CHUNK,), jnp.int32),
            pltpu.VMEM((CHUNK,), jnp.int32),
            pltpu.VMEM((CHUNK, cs), jnp.float32),
            pltpu.VMEM_SHARED((N_ACC, cs), jnp.float32),
            pltpu.SemaphoreType.DMA,
        ],
    )


_prop128 = _make_prop(128)
_prop64 = _make_prop(64)


def _dinv_of(p_ref):
    return lax.rsqrt(p_ref[0, :, 0:1] + p_ref[1, :, 0:1])


def _mm1_body(x_ref, w_ref, p_ref, o_ref):
    dinv = _dinv_of(p_ref)
    v = jnp.dot(x_ref[...], w_ref[...], preferred_element_type=jnp.float32)
    v = v * dinv
    o_ref[0] = v[:, :128]
    o_ref[1] = v[:, 128:]


def _mm2_body(s1_ref, p_ref, b_ref, w_ref, o_ref):
    dinv = _dinv_of(p_ref)
    h = jnp.concatenate([s1_ref[0], s1_ref[1]], axis=1) * dinv + b_ref[...]
    h = jnp.maximum(h, 0.0)
    v = jnp.dot(h, w_ref[...], preferred_element_type=jnp.float32) * dinv
    o_ref[0] = v[:, :64]
    o_ref[1] = v[:, 64:]


def _z_body(s2_ref, p_ref, b_ref, z_ref, zs_ref):
    dinv = _dinv_of(p_ref)
    z = jnp.concatenate([s2_ref[0], s2_ref[1]], axis=1) * dinv + b_ref[...]
    z_ref[...] = z
    v = z * dinv
    zs_ref[0] = v[:, :64]
    zs_ref[1] = v[:, 64:]


def _out_body(s3_ref, p_ref, w_ref, b_ref, o_ref):
    dinv = _dinv_of(p_ref)
    pz = jnp.concatenate([s3_ref[0], s3_ref[1]], axis=1) * dinv
    o_ref[...] = (jnp.dot(pz, w_ref[...], preferred_element_type=jnp.float32)
                  + b_ref[...])


def _p_spec():
    return pl.BlockSpec((2, BM, 16), lambda i: (0, i, 0))


def _mm1(x, w1, p):
    return pl.pallas_call(
        _mm1_body,
        grid=(N // BM,),
        in_specs=[pl.BlockSpec((BM, 256), lambda i: (i, 0)),
                  pl.BlockSpec((256, 256), lambda i: (0, 0)),
                  _p_spec()],
        out_specs=pl.BlockSpec((2, BM, 128), lambda i: (0, i, 0)),
        out_shape=jax.ShapeDtypeStruct((2, N, 128), jnp.float32),
    )(x, w1, p)


def _mm2(s1, p, b1, w2):
    return pl.pallas_call(
        _mm2_body,
        grid=(N // BM,),
        in_specs=[pl.BlockSpec((2, BM, 128), lambda i: (0, i, 0)),
                  _p_spec(),
                  pl.BlockSpec((1, 256), lambda i: (0, 0)),
                  pl.BlockSpec((256, 128), lambda i: (0, 0))],
        out_specs=pl.BlockSpec((2, BM, 64), lambda i: (0, i, 0)),
        out_shape=jax.ShapeDtypeStruct((2, N, 64), jnp.float32),
    )(s1, p, b1, w2)


def _zk(s2, p, b2):
    return pl.pallas_call(
        _z_body,
        grid=(N // BM,),
        in_specs=[pl.BlockSpec((2, BM, 64), lambda i: (0, i, 0)),
                  _p_spec(),
                  pl.BlockSpec((1, 128), lambda i: (0, 0))],
        out_specs=[pl.BlockSpec((BM, 128), lambda i: (i, 0)),
                   pl.BlockSpec((2, BM, 64), lambda i: (0, i, 0))],
        out_shape=[jax.ShapeDtypeStruct((N, 128), jnp.float32),
                   jax.ShapeDtypeStruct((2, N, 64), jnp.float32)],
    )(s2, p, b2)


def _outk(s3, p, w3, b3):
    return pl.pallas_call(
        _out_body,
        grid=(N // BM,),
        in_specs=[pl.BlockSpec((2, BM, 64), lambda i: (0, i, 0)),
                  _p_spec(),
                  pl.BlockSpec((128, 256), lambda i: (0, 0)),
                  pl.BlockSpec((1, 256), lambda i: (0, 0))],
        out_specs=pl.BlockSpec((BM, 256), lambda i: (i, 0)),
        out_shape=jax.ShapeDtypeStruct((N, 256), jnp.float32),
    )(s3, p, w3, b3)


def kernel(x, edge_index, W1, b1, W2, b2, W3, b3):
    src = edge_index[0].astype(jnp.int32)
    dst = edge_index[1].astype(jnp.int32)
    pad = E_PAD - E
    src_p = jnp.concatenate([src, jnp.zeros((pad,), jnp.int32)])
    dst_p = jnp.concatenate([dst, jnp.full((pad,), N, jnp.int32)])
    vals = jnp.stack([jnp.ones((CHUNK, 16), jnp.float32),
                      jnp.zeros((CHUNK, 16), jnp.float32)])

    p = _deg(dst_p, vals).reshape(2, N, 16)
    y1 = _mm1(x, W1, p)
    s1 = _prop128(y1.reshape(2 * N, 128), src_p, dst_p).reshape(2, N, 128)
    y2 = _mm2(s1, p, b1.reshape(1, -1), W2)
    s2 = _prop64(y2.reshape(2 * N, 64), src_p, dst_p).reshape(2, N, 64)
    z, zs = _zk(s2, p, b2.reshape(1, -1))
    s3 = _prop64(zs.reshape(2 * N, 64), src_p, dst_p).reshape(2, N, 64)
    out = _outk(s3, p, W3, b3.reshape(1, -1))
    return (out, z)


# speedup vs baseline: 6.1200x; 6.1200x over previous
import functools

import jax
import jax.numpy as jnp
from jax import lax
from jax.experimental import pallas as pl
from jax.experimental.pallas import tpu as pltpu
from jax.experimental.pallas import tpu_sc as plsc

N = 10000
E = 160000
E_PAD = 163840
CHUNK = 128
N_ACC = 10016
NT = 16
RPT = 632
BM = 400

_MESH = dict(core_axis_name="c", subcore_axis_name="s")


def _per_tile_rows(s, fn):
    @pl.when(s < NT - 1)
    def _():
        fn(s * RPT, RPT)

    @pl.when(s == NT - 1)
    def _():
        fn((NT - 1) * RPT, N - (NT - 1) * RPT)


def _deg_body(dst_hbm, vals_hbm, out_hbm, idx_d, ones_v, acc, sem):
    c = lax.axis_index("c")
    s = lax.axis_index("s")
    pltpu.sync_copy(vals_hbm.at[c], ones_v)
    for k in range(5):
        pltpu.sync_copy(ones_v, acc.at[pl.ds(s * 640 + k * CHUNK, CHUNK)])
    plsc.subcore_barrier()
    ept = E_PAD // (2 * NT)
    base0 = (c * NT + s) * ept

    def body(i, carry):
        pltpu.sync_copy(dst_hbm.at[pl.ds(base0 + i * CHUNK, CHUNK)], idx_d)
        pltpu.sync_copy(ones_v, acc.at[idx_d], add=True)
        return carry

    lax.fori_loop(0, ept // CHUNK, body, 0)
    plsc.subcore_barrier()

    def copy_out(off, nrows):
        pltpu.sync_copy(acc.at[pl.ds(off, nrows)],
                        out_hbm.at[pl.ds(c * N + off, nrows)])

    _per_tile_rows(s, copy_out)


@functools.partial(
    pl.kernel,
    out_type=jax.ShapeDtypeStruct((2 * N, 16), jnp.float32),
    mesh=plsc.VectorSubcoreMesh(**_MESH),
    scratch_types=[
        pltpu.VMEM((CHUNK,), jnp.int32),
        pltpu.VMEM((CHUNK, 16), jnp.float32),
        pltpu.VMEM_SHARED((10240, 16), jnp.float32),
        pltpu.SemaphoreType.DMA,
    ],
)
def _deg(dst_hbm, vals_hbm, out_hbm, idx_d, ones_v, acc, sem):
    _deg_body(dst_hbm, vals_hbm, out_hbm, idx_d, ones_v, acc, sem)


def _make_prop_split():
    cs = 128

    def body(y_hbm, src_hbm, dst_hbm, out_hbm, idx_s, idx_d, rows, acc, sem):
        c = lax.axis_index("c")
        s = lax.axis_index("s")
        coff = c * N

        def copy_in(off, nrows):
            pltpu.sync_copy(y_hbm.at[pl.ds(coff + off, nrows)],
                            acc.at[pl.ds(off, nrows)])

        _per_tile_rows(s, copy_in)
        plsc.subcore_barrier()
        ept = E_PAD // NT
        base0 = s * ept

        def step(i, carry):
            base = base0 + i * CHUNK
            pltpu.sync_copy(src_hbm.at[pl.ds(base, CHUNK)], idx_s)
            pltpu.sync_copy(dst_hbm.at[pl.ds(base, CHUNK)], idx_d)

            def addoff(j, cc):
                idx_s[pl.ds(j * 16, 16)] = idx_s[pl.ds(j * 16, 16)] + coff
                return cc

            lax.fori_loop(0, CHUNK // 16, addoff, 0)
            pltpu.async_copy(y_hbm.at[idx_s], rows, sem).wait()
            pltpu.sync_copy(rows, acc.at[idx_d], add=True)
            return carry

        lax.fori_loop(0, ept // CHUNK, step, 0)
        plsc.subcore_barrier()

        def copy_out(off, nrows):
            pltpu.sync_copy(acc.at[pl.ds(off, nrows)],
                            out_hbm.at[pl.ds(coff + off, nrows)])

        _per_tile_rows(s, copy_out)

    return pl.kernel(
        body,
        out_type=jax.ShapeDtypeStruct((2 * N, cs), jnp.float32),
        mesh=plsc.VectorSubcoreMesh(**_MESH),
        scratch_types=[
            pltpu.VMEM((CHUNK,), jnp.int32),
            pltpu.VMEM((CHUNK,), jnp.int32),
            pltpu.VMEM((CHUNK, cs), jnp.float32),
            pltpu.VMEM_SHARED((N_ACC, cs), jnp.float32),
            pltpu.SemaphoreType.DMA,
        ],
    )


def _make_prop_half():
    cs = 128

    def body(y_hbm, src_hbm, dst_hbm, out_hbm, idx_s, idx_d, rows, acc, sem):
        c = lax.axis_index("c")
        s = lax.axis_index("s")

        def copy_in(off, nrows):
            pltpu.sync_copy(y_hbm.at[pl.ds(off, nrows)],
                            acc.at[pl.ds(off, nrows)])

        def zero_in(off, nrows):
            for k in range(nrows // CHUNK):
                pltpu.sync_copy(rows.at[pl.ds(0, CHUNK)],
                                acc.at[pl.ds(off + k * CHUNK, CHUNK)])
            rem = nrows % CHUNK
            pltpu.sync_copy(rows.at[pl.ds(0, rem)],
                            acc.at[pl.ds(off + (nrows // CHUNK) * CHUNK,
                                         rem)])

        @pl.when(c == 0)
        def _():
            _per_tile_rows(s, copy_in)

        @pl.when(c == 1)
        def _():
            def zrow(j, cc):
                rows[j // 8, pl.ds((j % 8) * 16, 16)] = jnp.zeros(
                    (16,), jnp.float32)
                return cc

            lax.fori_loop(0, CHUNK * 8, zrow, 0)
            _per_tile_rows(s, zero_in)

        plsc.subcore_barrier()
        ept = E_PAD // (2 * NT)
        base0 = (c * NT + s) * ept

        def step(i, carry):
            base = base0 + i * CHUNK
            pltpu.sync_copy(src_hbm.at[pl.ds(base, CHUNK)], idx_s)
            pltpu.sync_copy(dst_hbm.at[pl.ds(base, CHUNK)], idx_d)
            pltpu.async_copy(y_hbm.at[idx_s], rows, sem).wait()
            pltpu.sync_copy(rows, acc.at[idx_d], add=True)
            return carry

        lax.fori_loop(0, ept // CHUNK, step, 0)
        plsc.subcore_barrier()

        def copy_out(off, nrows):
            pltpu.sync_copy(acc.at[pl.ds(off, nrows)],
                            out_hbm.at[pl.ds(c * N + off, nrows)])

        _per_tile_rows(s, copy_out)

    return pl.kernel(
        body,
        out_type=jax.ShapeDtypeStruct((2 * N, cs), jnp.float32),
        mesh=plsc.VectorSubcoreMesh(**_MESH),
        scratch_types=[
            pltpu.VMEM((CHUNK,), jnp.int32),
            pltpu.VMEM((CHUNK,), jnp.int32),
            pltpu.VMEM((CHUNK, cs), jnp.float32),
            pltpu.VMEM_SHARED((N_ACC, cs), jnp.float32),
            pltpu.SemaphoreType.DMA,
        ],
    )


_prop256 = _make_prop_split()
_prop128p = _make_prop_half()


def _dinv_of(p_ref):
    return lax.rsqrt(p_ref[0, :, 0:1] + p_ref[1, :, 0:1])


def _mm1_body(x_ref, w_ref, p_ref, o_ref):
    dinv = _dinv_of(p_ref)
    v = jnp.dot(x_ref[...], w_ref[...], preferred_element_type=jnp.float32)
    v = v * dinv
    o_ref[0] = v[:, :128]
    o_ref[1] = v[:, 128:]


def _mm2_body(s1_ref, p_ref, b_ref, w_ref, o_ref):
    dinv = _dinv_of(p_ref)
    h = jnp.concatenate([s1_ref[0], s1_ref[1]], axis=1) * dinv + b_ref[...]
    h = jnp.maximum(h, 0.0)
    o_ref[...] = jnp.dot(h, w_ref[...],
                         preferred_element_type=jnp.float32) * dinv


def _z_body(s2_ref, p_ref, b_ref, z_ref, zs_ref):
    dinv = _dinv_of(p_ref)
    z = (s2_ref[0] + s2_ref[1]) * dinv + b_ref[...]
    z_ref[...] = z
    zs_ref[...] = z * dinv


def _out_body(s3_ref, p_ref, w_ref, b_ref, o_ref):
    dinv = _dinv_of(p_ref)
    pz = (s3_ref[0] + s3_ref[1]) * dinv
    o_ref[...] = (jnp.dot(pz, w_ref[...], preferred_element_type=jnp.float32)
                  + b_ref[...])


def _p_spec():
    return pl.BlockSpec((2, BM, 16), lambda i: (0, i, 0))


def _mm1(x, w1, p):
    return pl.pallas_call(
        _mm1_body,
        grid=(N // BM,),
        in_specs=[pl.BlockSpec((BM, 256), lambda i: (i, 0)),
                  pl.BlockSpec((256, 256), lambda i: (0, 0)),
                  _p_spec()],
        out_specs=pl.BlockSpec((2, BM, 128), lambda i: (0, i, 0)),
        out_shape=jax.ShapeDtypeStruct((2, N, 128), jnp.float32),
    )(x, w1, p)


def _mm2(s1, p, b1, w2):
    return pl.pallas_call(
        _mm2_body,
        grid=(N // BM,),
        in_specs=[pl.BlockSpec((2, BM, 128), lambda i: (0, i, 0)),
                  _p_spec(),
                  pl.BlockSpec((1, 256), lambda i: (0, 0)),
                  pl.BlockSpec((256, 128), lambda i: (0, 0))],
        out_specs=pl.BlockSpec((BM, 128), lambda i: (i, 0)),
        out_shape=jax.ShapeDtypeStruct((N, 128), jnp.float32),
    )(s1, p, b1, w2)


def _zk(s2, p, b2):
    return pl.pallas_call(
        _z_body,
        grid=(N // BM,),
        in_specs=[pl.BlockSpec((2, BM, 128), lambda i: (0, i, 0)),
                  _p_spec(),
                  pl.BlockSpec((1, 128), lambda i: (0, 0))],
        out_specs=[pl.BlockSpec((BM, 128), lambda i: (i, 0)),
                   pl.BlockSpec((BM, 128), lambda i: (i, 0))],
        out_shape=[jax.ShapeDtypeStruct((N, 128), jnp.float32),
                   jax.ShapeDtypeStruct((N, 128), jnp.float32)],
    )(s2, p, b2)


def _outk(s3, p, w3, b3):
    return pl.pallas_call(
        _out_body,
        grid=(N // BM,),
        in_specs=[pl.BlockSpec((2, BM, 128), lambda i: (0, i, 0)),
                  _p_spec(),
                  pl.BlockSpec((128, 256), lambda i: (0, 0)),
                  pl.BlockSpec((1, 256), lambda i: (0, 0))],
        out_specs=pl.BlockSpec((BM, 256), lambda i: (i, 0)),
        out_shape=jax.ShapeDtypeStruct((N, 256), jnp.float32),
    )(s3, p, w3, b3)


def kernel(x, edge_index, W1, b1, W2, b2, W3, b3):
    src = edge_index[0].astype(jnp.int32)
    dst = edge_index[1].astype(jnp.int32)
    pad = E_PAD - E
    src_p = jnp.concatenate([src, jnp.zeros((pad,), jnp.int32)])
    dst_p = jnp.concatenate([dst, jnp.full((pad,), N, jnp.int32)])
    vals = jnp.stack([jnp.ones((CHUNK, 16), jnp.float32),
                      jnp.zeros((CHUNK, 16), jnp.float32)])

    p = _deg(dst_p, vals).reshape(2, N, 16)
    y1 = _mm1(x, W1, p)
    s1 = _prop256(y1.reshape(2 * N, 128), src_p, dst_p).reshape(2, N, 128)
    y2 = _mm2(s1, p, b1.reshape(1, -1), W2)
    s2 = _prop128p(y2, src_p, dst_p).reshape(2, N, 128)
    z, zs = _zk(s2, p, b2.reshape(1, -1))
    s3 = _prop128p(zs, src_p, dst_p).reshape(2, N, 128)
    out = _outk(s3, p, W3, b3.reshape(1, -1))
    return (out, z)


# speedup vs baseline: 6.9885x; 1.1419x over previous
import functools

import jax
import jax.numpy as jnp
from jax import lax
from jax.experimental import pallas as pl
from jax.experimental.pallas import tpu as pltpu
from jax.experimental.pallas import tpu_sc as plsc

N = 10000
E = 160000
E_PAD = 163840
CHUNK = 128
N_ACC = 10016
NT = 16
RPT = 632
BM = 400

_MESH = dict(core_axis_name="c", subcore_axis_name="s")


def _per_tile_rows(s, fn):
    @pl.when(s < NT - 1)
    def _():
        fn(s * RPT, RPT)

    @pl.when(s == NT - 1)
    def _():
        fn((NT - 1) * RPT, N - (NT - 1) * RPT)


def _deg_body(dst_hbm, vals_hbm, out_hbm, idx_d, ones_v, acc, sem):
    c = lax.axis_index("c")
    s = lax.axis_index("s")
    pltpu.sync_copy(vals_hbm.at[c], ones_v)
    for k in range(5):
        pltpu.sync_copy(ones_v, acc.at[pl.ds(s * 640 + k * CHUNK, CHUNK)])
    plsc.subcore_barrier()
    ept = E_PAD // (2 * NT)
    base0 = (c * NT + s) * ept

    def body(i, carry):
        pltpu.sync_copy(dst_hbm.at[pl.ds(base0 + i * CHUNK, CHUNK)], idx_d)
        pltpu.sync_copy(ones_v, acc.at[idx_d], add=True)
        return carry

    lax.fori_loop(0, ept // CHUNK, body, 0)
    plsc.subcore_barrier()

    def copy_out(off, nrows):
        pltpu.sync_copy(acc.at[pl.ds(off, nrows)],
                        out_hbm.at[pl.ds(c * N + off, nrows)])

    _per_tile_rows(s, copy_out)


@functools.partial(
    pl.kernel,
    out_type=jax.ShapeDtypeStruct((2 * N, 16), jnp.float32),
    mesh=plsc.VectorSubcoreMesh(**_MESH),
    scratch_types=[
        pltpu.VMEM((CHUNK,), jnp.int32),
        pltpu.VMEM((CHUNK, 16), jnp.float32),
        pltpu.VMEM_SHARED((10240, 16), jnp.float32),
        pltpu.SemaphoreType.DMA,
    ],
)
def _deg(dst_hbm, vals_hbm, out_hbm, idx_d, ones_v, acc, sem):
    _deg_body(dst_hbm, vals_hbm, out_hbm, idx_d, ones_v, acc, sem)


NB = 2


def _make_prop(split):
    nw = (E_PAD // (NT * CHUNK)) if split else (E_PAD // (2 * NT * CHUNK))

    phases = (40, 40) if split else (24, 16)
    nbuf = max(phases)

    def body(y_hbm, src_hbm, dst_hbm, out_hbm, srcb, dstb, r0, r1, acc,
             *sems):
        rows = (r0, r1)
        gsem = sems[:NB]
        c = lax.axis_index("c")
        s = lax.axis_index("s")
        srow = (c * NT + s) * nw
        drow = s * nw if split else srow

        zbuf = rows[0]

        def zrow(j, cc):
            zbuf[j // 8, pl.ds((j % 8) * 16, 16)] = jnp.zeros(
                (16,), jnp.float32)
            return cc

        lax.fori_loop(0, CHUNK * 8, zrow, 0)

        def zero_in(off, nrows):
            nfull = nrows // CHUNK
            for k in range(nfull):
                pltpu.sync_copy(zbuf, acc.at[pl.ds(off + k * CHUNK, CHUNK)])
            rem = nrows - nfull * CHUNK
            pltpu.sync_copy(zbuf.at[pl.ds(0, rem)],
                            acc.at[pl.ds(off + nfull * CHUNK, rem)])

        _per_tile_rows(s, zero_in)

        def gstart(w, par):
            pltpu.async_copy(y_hbm.at[srcb.at[w]], rows[par], gsem[par])

        def gwait(w, par):
            pltpu.make_async_copy(y_hbm.at[srcb.at[w]], rows[par],
                                  gsem[par]).wait()

        def sdo(w, par):
            pltpu.sync_copy(rows[par], acc.at[dstb.at[w]], add=True)

        plsc.subcore_barrier()

        def run_phase(off, cnt):
            pltpu.sync_copy(src_hbm.at[pl.ds(srow + off, cnt)],
                            srcb.at[pl.ds(0, cnt)])
            pltpu.sync_copy(dst_hbm.at[pl.ds(drow + off, cnt)],
                            dstb.at[pl.ds(0, cnt)])
            gstart(0, 0)

            def outer(wo, cc):
                for par in range(NB):
                    w = wo * NB + par
                    gwait(w, par)

                    @pl.when(w + 1 < cnt)
                    def _():
                        gstart(w + 1, (par + 1) % NB)

                    sdo(w, par)
                return cc

            lax.fori_loop(0, cnt // NB, outer, 0)

        off = 0
        for cnt in phases:
            run_phase(off, cnt)
            off += cnt
        plsc.subcore_barrier()

        def copy_out(off, nrows):
            pltpu.sync_copy(acc.at[pl.ds(off, nrows)],
                            out_hbm.at[pl.ds(c * N + off, nrows)])

        _per_tile_rows(s, copy_out)

    return pl.kernel(
        body,
        out_type=jax.ShapeDtypeStruct((2 * N, 128), jnp.float32),
        mesh=plsc.VectorSubcoreMesh(**_MESH),
        scratch_types=(
            [pltpu.VMEM((max((40, 40) if split else (24, 16)), CHUNK),
                        jnp.int32)] * 2
            + [pltpu.VMEM((CHUNK, 128), jnp.float32)] * NB
            + [pltpu.VMEM_SHARED((N_ACC, 128), jnp.float32)]
            + [pltpu.SemaphoreType.DMA] * NB
        ),
    )


_prop256 = _make_prop(True)
_prop128p = _make_prop(False)


def _dinv_of(p_ref):
    return lax.rsqrt(p_ref[0, :, 0:1] + p_ref[1, :, 0:1])


def _mm1_body(x_ref, w_ref, p_ref, o_ref):
    dinv = _dinv_of(p_ref)
    v = jnp.dot(x_ref[...], w_ref[...], preferred_element_type=jnp.float32)
    v = v * dinv
    o_ref[0] = v[:, :128]
    o_ref[1] = v[:, 128:]


def _mm2_body(s1_ref, y1_ref, p_ref, b_ref, w_ref, o_ref):
    dinv = _dinv_of(p_ref)
    h = (jnp.concatenate([s1_ref[0] + y1_ref[0], s1_ref[1] + y1_ref[1]],
                         axis=1) * dinv + b_ref[...])
    h = jnp.maximum(h, 0.0)
    o_ref[...] = jnp.dot(h, w_ref[...],
                         preferred_element_type=jnp.float32) * dinv


def _z_body(s2_ref, y2_ref, p_ref, b_ref, z_ref, zs_ref):
    dinv = _dinv_of(p_ref)
    z = (s2_ref[0] + s2_ref[1] + y2_ref[...]) * dinv + b_ref[...]
    z_ref[...] = z
    zs_ref[...] = z * dinv


def _out_body(s3_ref, zs_ref, p_ref, w_ref, b_ref, o_ref):
    dinv = _dinv_of(p_ref)
    pz = (s3_ref[0] + s3_ref[1] + zs_ref[...]) * dinv
    o_ref[...] = (jnp.dot(pz, w_ref[...], preferred_element_type=jnp.float32)
                  + b_ref[...])


def _p_spec():
    return pl.BlockSpec((2, BM, 16), lambda i: (0, i, 0))


def _mm1(x, w1, p):
    return pl.pallas_call(
        _mm1_body,
        grid=(N // BM,),
        in_specs=[pl.BlockSpec((BM, 256), lambda i: (i, 0)),
                  pl.BlockSpec((256, 256), lambda i: (0, 0)),
                  _p_spec()],
        out_specs=pl.BlockSpec((2, BM, 128), lambda i: (0, i, 0)),
        out_shape=jax.ShapeDtypeStruct((2, N, 128), jnp.float32),
    )(x, w1, p)


def _mm2(s1, y1, p, b1, w2):
    return pl.pallas_call(
        _mm2_body,
        grid=(N // BM,),
        in_specs=[pl.BlockSpec((2, BM, 128), lambda i: (0, i, 0)),
                  pl.BlockSpec((2, BM, 128), lambda i: (0, i, 0)),
                  _p_spec(),
                  pl.BlockSpec((1, 256), lambda i: (0, 0)),
                  pl.BlockSpec((256, 128), lambda i: (0, 0))],
        out_specs=pl.BlockSpec((BM, 128), lambda i: (i, 0)),
        out_shape=jax.ShapeDtypeStruct((N, 128), jnp.float32),
    )(s1, y1, p, b1, w2)


def _zk(s2, y2, p, b2):
    return pl.pallas_call(
        _z_body,
        grid=(N // BM,),
        in_specs=[pl.BlockSpec((2, BM, 128), lambda i: (0, i, 0)),
                  pl.BlockSpec((BM, 128), lambda i: (i, 0)),
                  _p_spec(),
                  pl.BlockSpec((1, 128), lambda i: (0, 0))],
        out_specs=[pl.BlockSpec((BM, 128), lambda i: (i, 0)),
                   pl.BlockSpec((BM, 128), lambda i: (i, 0))],
        out_shape=[jax.ShapeDtypeStruct((N, 128), jnp.float32),
                   jax.ShapeDtypeStruct((N, 128), jnp.float32)],
    )(s2, y2, p, b2)


def _outk(s3, zs, p, w3, b3):
    return pl.pallas_call(
        _out_body,
        grid=(N // BM,),
        in_specs=[pl.BlockSpec((2, BM, 128), lambda i: (0, i, 0)),
                  pl.BlockSpec((BM, 128), lambda i: (i, 0)),
                  _p_spec(),
                  pl.BlockSpec((128, 256), lambda i: (0, 0)),
                  pl.BlockSpec((1, 256), lambda i: (0, 0))],
        out_specs=pl.BlockSpec((BM, 256), lambda i: (i, 0)),
        out_shape=jax.ShapeDtypeStruct((N, 256), jnp.float32),
    )(s3, zs, p, w3, b3)


def kernel(x, edge_index, W1, b1, W2, b2, W3, b3):
    src = edge_index[0].astype(jnp.int32)
    dst = edge_index[1].astype(jnp.int32)
    pad = E_PAD - E
    src_p = jnp.concatenate([src, jnp.zeros((pad,), jnp.int32)])
    dst_p = jnp.concatenate([dst, jnp.full((pad,), N, jnp.int32)])
    src2 = src_p.reshape(-1, CHUNK)
    srcB = jnp.concatenate([src_p, src_p + N]).reshape(-1, CHUNK)
    dst2 = dst_p.reshape(-1, CHUNK)
    vals = jnp.stack([jnp.ones((CHUNK, 16), jnp.float32),
                      jnp.zeros((CHUNK, 16), jnp.float32)])

    p = _deg(dst_p, vals).reshape(2, N, 16)
    y1 = _mm1(x, W1, p)
    s1 = _prop256(y1.reshape(2 * N, 128), srcB, dst2).reshape(2, N, 128)
    y2 = _mm2(s1, y1, p, b1.reshape(1, -1), W2)
    s2 = _prop128p(y2, src2, dst2).reshape(2, N, 128)
    z, zs = _zk(s2, y2, p, b2.reshape(1, -1))
    s3 = _prop128p(zs, src2, dst2).reshape(2, N, 128)
    out = _outk(s3, zs, p, W3, b3.reshape(1, -1))
    return (out, z)


# speedup vs baseline: 16.2311x; 2.3225x over previous
import functools

import jax
import jax.numpy as jnp
from jax import lax
from jax.experimental import pallas as pl
from jax.experimental.pallas import tpu as pltpu
from jax.experimental.pallas import tpu_sc as plsc

N = 10000
E = 160000
E_PAD = 163840
CHUNK = 128
N_ACC = 10240
NT = 16
RPT = 632
BM = 400

_MESH = dict(core_axis_name="c", subcore_axis_name="s")


def _per_tile_rows(s, fn):
    @pl.when(s < NT - 1)
    def _():
        fn(s * RPT, RPT)

    @pl.when(s == NT - 1)
    def _():
        fn((NT - 1) * RPT, N - (NT - 1) * RPT)


def _deg_body(dst_hbm, vals_hbm, out_hbm, idx_d, ones_v, acc, sem):
    c = lax.axis_index("c")
    s = lax.axis_index("s")
    pltpu.sync_copy(vals_hbm.at[c], ones_v)
    for k in range(5):
        pltpu.sync_copy(ones_v, acc.at[pl.ds(s * 640 + k * CHUNK, CHUNK)])
    plsc.subcore_barrier()
    ept = E_PAD // (2 * NT)
    base0 = (c * NT + s) * ept

    def body(i, carry):
        pltpu.sync_copy(dst_hbm.at[pl.ds(base0 + i * CHUNK, CHUNK)], idx_d)
        pltpu.sync_copy(ones_v, acc.at[idx_d], add=True)
        return carry

    lax.fori_loop(0, ept // CHUNK, body, 0)
    plsc.subcore_barrier()

    def copy_out(off, nrows):
        pltpu.sync_copy(acc.at[pl.ds(off, nrows)],
                        out_hbm.at[pl.ds(c * N + off, nrows)])

    _per_tile_rows(s, copy_out)


@functools.partial(
    pl.kernel,
    out_type=jax.ShapeDtypeStruct((2 * N, 16), jnp.float32),
    mesh=plsc.VectorSubcoreMesh(**_MESH),
    scratch_types=[
        pltpu.VMEM((CHUNK,), jnp.int32),
        pltpu.VMEM((CHUNK, 16), jnp.float32),
        pltpu.VMEM_SHARED((10240, 16), jnp.float32),
        pltpu.SemaphoreType.DMA,
    ],
)
def _deg(dst_hbm, vals_hbm, out_hbm, idx_d, ones_v, acc, sem):
    _deg_body(dst_hbm, vals_hbm, out_hbm, idx_d, ones_v, acc, sem)


NB = 2


def _make_prop(split):
    nw = (E_PAD // (NT * CHUNK)) if split else (E_PAD // (2 * NT * CHUNK))

    phases = (40, 40) if split else (24, 16)
    nbuf = max(phases)

    def body(y_hbm, src_hbm, dst_hbm, out_hbm, srcb, dstb, r0, r1, acc,
             *sems):
        rows = (r0, r1)
        gsem = sems[:NB]
        c = lax.axis_index("c")
        s = lax.axis_index("s")
        srow = (c * NT + s) * nw
        drow = s * nw if split else srow

        zbuf = rows[0]

        def zrow(j, cc):
            zbuf[j // 8, pl.ds((j % 8) * 16, 16)] = jnp.zeros(
                (16,), jnp.float32)
            return cc

        lax.fori_loop(0, CHUNK * 8, zrow, 0)

        def zero_in(off, nrows):
            nfull = nrows // CHUNK
            for k in range(nfull):
                pltpu.sync_copy(zbuf, acc.at[pl.ds(off + k * CHUNK, CHUNK)])
            rem = nrows - nfull * CHUNK
            pltpu.sync_copy(zbuf.at[pl.ds(0, rem)],
                            acc.at[pl.ds(off + nfull * CHUNK, rem)])

        _per_tile_rows(s, zero_in)

        def gstart(w, par):
            pltpu.async_copy(y_hbm.at[srcb.at[w]], rows[par], gsem[par])

        def gwait(w, par):
            pltpu.make_async_copy(y_hbm.at[srcb.at[w]], rows[par],
                                  gsem[par]).wait()

        def sdo(w, par):
            pltpu.sync_copy(rows[par], acc.at[dstb.at[w]], add=True)

        plsc.subcore_barrier()

        def run_phase(off, cnt):
            pltpu.sync_copy(src_hbm.at[pl.ds(srow + off, cnt)],
                            srcb.at[pl.ds(0, cnt)])
            pltpu.sync_copy(dst_hbm.at[pl.ds(drow + off, cnt)],
                            dstb.at[pl.ds(0, cnt)])
            gstart(0, 0)

            def outer(wo, cc):
                for par in range(NB):
                    w = wo * NB + par
                    gwait(w, par)

                    @pl.when(w + 1 < cnt)
                    def _():
                        gstart(w + 1, (par + 1) % NB)

                    sdo(w, par)
                return cc

            lax.fori_loop(0, cnt // NB, outer, 0)

        off = 0
        for cnt in phases:
            run_phase(off, cnt)
            off += cnt
        plsc.subcore_barrier()

        def copy_out(off, nrows):
            pltpu.sync_copy(acc.at[pl.ds(off, nrows)],
                            out_hbm.at[pl.ds(c * N + off, nrows)])

        _per_tile_rows(s, copy_out)

    return pl.kernel(
        body,
        out_type=jax.ShapeDtypeStruct((2 * N, 128), jnp.float32),
        mesh=plsc.VectorSubcoreMesh(**_MESH),
        scratch_types=(
            [pltpu.VMEM((max((40, 40) if split else (24, 16)), CHUNK),
                        jnp.int32)] * 2
            + [pltpu.VMEM((CHUNK, 128), jnp.float32)] * NB
            + [pltpu.VMEM_SHARED((N_ACC, 128), jnp.float32)]
            + [pltpu.SemaphoreType.DMA] * NB
        ),
    )


_prop256 = _make_prop(True)
_prop128p = _make_prop(False)


def _dinv_of(p_ref):
    return lax.rsqrt(p_ref[0, :, 0:1] + p_ref[1, :, 0:1])


def _mm1_body(x_ref, w_ref, p_ref, o_ref):
    dinv = _dinv_of(p_ref)
    v = jnp.dot(x_ref[...], w_ref[...], preferred_element_type=jnp.float32)
    v = v * dinv
    o_ref[0] = v[:, :128]
    o_ref[1] = v[:, 128:]


def _mm2_body(s1_ref, y1_ref, p_ref, b_ref, w_ref, o_ref):
    dinv = _dinv_of(p_ref)
    h = (jnp.concatenate([s1_ref[0] + y1_ref[0], s1_ref[1] + y1_ref[1]],
                         axis=1) * dinv + b_ref[...])
    h = jnp.maximum(h, 0.0)
    o_ref[...] = jnp.dot(h, w_ref[...],
                         preferred_element_type=jnp.float32) * dinv


def _z_body(s2_ref, y2_ref, p_ref, b_ref, z_ref, zs_ref):
    dinv = _dinv_of(p_ref)
    z = (s2_ref[0] + s2_ref[1] + y2_ref[...]) * dinv + b_ref[...]
    z_ref[...] = z
    zs_ref[...] = z * dinv


def _out_body(s3_ref, zs_ref, p_ref, w_ref, b_ref, o_ref):
    dinv = _dinv_of(p_ref)
    pz = (s3_ref[0] + s3_ref[1] + zs_ref[...]) * dinv
    o_ref[...] = (jnp.dot(pz, w_ref[...], preferred_element_type=jnp.float32)
                  + b_ref[...])


def _p_spec():
    return pl.BlockSpec((2, BM, 16), lambda i: (0, i, 0))


def _mm1(x, w1, p):
    return pl.pallas_call(
        _mm1_body,
        grid=(N // BM,),
        in_specs=[pl.BlockSpec((BM, 256), lambda i: (i, 0)),
                  pl.BlockSpec((256, 256), lambda i: (0, 0)),
                  _p_spec()],
        out_specs=pl.BlockSpec((2, BM, 128), lambda i: (0, i, 0)),
        out_shape=jax.ShapeDtypeStruct((2, N, 128), jnp.float32),
    )(x, w1, p)


def _mm2(s1, y1, p, b1, w2):
    return pl.pallas_call(
        _mm2_body,
        grid=(N // BM,),
        in_specs=[pl.BlockSpec((2, BM, 128), lambda i: (0, i, 0)),
                  pl.BlockSpec((2, BM, 128), lambda i: (0, i, 0)),
                  _p_spec(),
                  pl.BlockSpec((1, 256), lambda i: (0, 0)),
                  pl.BlockSpec((256, 128), lambda i: (0, 0))],
        out_specs=pl.BlockSpec((BM, 128), lambda i: (i, 0)),
        out_shape=jax.ShapeDtypeStruct((N, 128), jnp.float32),
    )(s1, y1, p, b1, w2)


def _zk(s2, y2, p, b2):
    return pl.pallas_call(
        _z_body,
        grid=(N // BM,),
        in_specs=[pl.BlockSpec((2, BM, 128), lambda i: (0, i, 0)),
                  pl.BlockSpec((BM, 128), lambda i: (i, 0)),
                  _p_spec(),
                  pl.BlockSpec((1, 128), lambda i: (0, 0))],
        out_specs=[pl.BlockSpec((BM, 128), lambda i: (i, 0)),
                   pl.BlockSpec((BM, 128), lambda i: (i, 0))],
        out_shape=[jax.ShapeDtypeStruct((N, 128), jnp.float32),
                   jax.ShapeDtypeStruct((N, 128), jnp.float32)],
    )(s2, y2, p, b2)


def _outk(s3, zs, p, w3, b3):
    return pl.pallas_call(
        _out_body,
        grid=(N // BM,),
        in_specs=[pl.BlockSpec((2, BM, 128), lambda i: (0, i, 0)),
                  pl.BlockSpec((BM, 128), lambda i: (i, 0)),
                  _p_spec(),
                  pl.BlockSpec((128, 256), lambda i: (0, 0)),
                  pl.BlockSpec((1, 256), lambda i: (0, 0))],
        out_specs=pl.BlockSpec((BM, 256), lambda i: (i, 0)),
        out_shape=jax.ShapeDtypeStruct((N, 256), jnp.float32),
    )(s3, zs, p, w3, b3)


def kernel(x, edge_index, W1, b1, W2, b2, W3, b3):
    src = edge_index[0].astype(jnp.int32)
    dst = edge_index[1].astype(jnp.int32)
    pad = E_PAD - E
    fill = jnp.arange(pad, dtype=jnp.int32)
    src_p = jnp.concatenate([src, fill % N])
    dst_p = jnp.concatenate([dst, N + fill % (N_ACC - N)])
    src2 = src_p.reshape(-1, CHUNK)
    srcB = jnp.concatenate([src_p, src_p + N]).reshape(-1, CHUNK)
    dst2 = dst_p.reshape(-1, CHUNK)
    vals = jnp.stack([jnp.ones((CHUNK, 16), jnp.float32),
                      jnp.zeros((CHUNK, 16), jnp.float32)])

    p = _deg(dst_p, vals).reshape(2, N, 16)
    y1 = _mm1(x, W1, p)
    s1 = _prop256(y1.reshape(2 * N, 128), srcB, dst2).reshape(2, N, 128)
    y2 = _mm2(s1, y1, p, b1.reshape(1, -1), W2)
    s2 = _prop128p(y2, src2, dst2).reshape(2, N, 128)
    z, zs = _zk(s2, y2, p, b2.reshape(1, -1))
    s3 = _prop128p(zs, src2, dst2).reshape(2, N, 128)
    out = _outk(s3, zs, p, W3, b3.reshape(1, -1))
    return (out, z)


# speedup vs baseline: 16.9864x; 1.0465x over previous
import functools

import jax
import jax.numpy as jnp
from jax import lax
from jax.experimental import pallas as pl
from jax.experimental.pallas import tpu as pltpu
from jax.experimental.pallas import tpu_sc as plsc

N = 10000
E = 160000
E_PAD = 163840
CHUNK = 128
N_ACC = 10240
NT = 16
RPT = 632
BM = 400

_MESH = dict(core_axis_name="c", subcore_axis_name="s")


def _per_tile_rows(s, fn):
    @pl.when(s < NT - 1)
    def _():
        fn(s * RPT, RPT)

    @pl.when(s == NT - 1)
    def _():
        fn((NT - 1) * RPT, N - (NT - 1) * RPT)


def _deg_body(dst_hbm, vals_hbm, out_hbm, dstb, ones_v, acc, sem):
    c = lax.axis_index("c")
    s = lax.axis_index("s")
    nw = E_PAD // (2 * NT * CHUNK)
    pltpu.sync_copy(vals_hbm.at[c], ones_v)
    pltpu.sync_copy(dst_hbm.at[pl.ds((c * NT + s) * nw, nw)], dstb)
    for k in range(5):
        pltpu.sync_copy(ones_v, acc.at[pl.ds(s * 640 + k * CHUNK, CHUNK)])
    plsc.subcore_barrier()

    def fire(w, carry):
        pltpu.async_copy(ones_v, acc.at[dstb.at[w]], sem, add=True)
        return carry

    def drain(w, carry):
        pltpu.make_async_copy(ones_v, acc.at[dstb.at[w]], sem).wait()
        return carry

    lax.fori_loop(0, nw, fire, 0)
    lax.fori_loop(0, nw, drain, 0)
    plsc.subcore_barrier()

    def copy_out(off, nrows):
        pltpu.sync_copy(acc.at[pl.ds(off, nrows)],
                        out_hbm.at[pl.ds(c * N + off, nrows)])

    _per_tile_rows(s, copy_out)


@functools.partial(
    pl.kernel,
    out_type=jax.ShapeDtypeStruct((2 * N, 16), jnp.float32),
    mesh=plsc.VectorSubcoreMesh(**_MESH),
    scratch_types=[
        pltpu.VMEM((E_PAD // (2 * NT * CHUNK), CHUNK), jnp.int32),
        pltpu.VMEM((CHUNK, 16), jnp.float32),
        pltpu.VMEM_SHARED((10240, 16), jnp.float32),
        pltpu.SemaphoreType.DMA,
    ],
)
def _deg(dst_hbm, vals_hbm, out_hbm, dstb, ones_v, acc, sem):
    _deg_body(dst_hbm, vals_hbm, out_hbm, dstb, ones_v, acc, sem)


NB = 2


def _make_prop(split):
    nw = (E_PAD // (NT * CHUNK)) if split else (E_PAD // (2 * NT * CHUNK))

    phases = (40, 40) if split else (24, 16)
    nbuf = max(phases)

    def body(y_hbm, src_hbm, dst_hbm, out_hbm, srcb, dstb, r0, r1, acc,
             *sems):
        rows = (r0, r1)
        gsem = sems[:NB]
        ssem = sems[NB:]
        c = lax.axis_index("c")
        s = lax.axis_index("s")
        srow = (c * NT + s) * nw
        drow = s * nw if split else srow

        zbuf = rows[0]

        def zrow(j, cc):
            zbuf[j // 8, pl.ds((j % 8) * 16, 16)] = jnp.zeros(
                (16,), jnp.float32)
            return cc

        lax.fori_loop(0, CHUNK * 8, zrow, 0)

        def zero_in(off, nrows):
            nfull = nrows // CHUNK
            for k in range(nfull):
                pltpu.sync_copy(zbuf, acc.at[pl.ds(off + k * CHUNK, CHUNK)])
            rem = nrows - nfull * CHUNK
            pltpu.sync_copy(zbuf.at[pl.ds(0, rem)],
                            acc.at[pl.ds(off + nfull * CHUNK, rem)])

        _per_tile_rows(s, zero_in)

        def gstart(w, par):
            pltpu.async_copy(y_hbm.at[srcb.at[w]], rows[par], gsem[par])

        def gwait(w, par):
            pltpu.make_async_copy(y_hbm.at[srcb.at[w]], rows[par],
                                  gsem[par]).wait()

        def sstart(w, par):
            pltpu.async_copy(rows[par], acc.at[dstb.at[w]], ssem[par],
                             add=True)

        def swait(w, par):
            pltpu.make_async_copy(rows[par], acc.at[dstb.at[w]],
                                  ssem[par]).wait()

        plsc.subcore_barrier()

        def run_phase(off, cnt):
            pltpu.sync_copy(src_hbm.at[pl.ds(srow + off, cnt)],
                            srcb.at[pl.ds(0, cnt)])
            pltpu.sync_copy(dst_hbm.at[pl.ds(drow + off, cnt)],
                            dstb.at[pl.ds(0, cnt)])
            gstart(0, 0)

            def outer(wo, cc):
                for par in range(NB):
                    w = wo * NB + par
                    gwait(w, par)

                    @pl.when(w + 1 < cnt)
                    def _():
                        @pl.when(w >= 1)
                        def _():
                            swait(w - 1, (par + 1) % NB)

                        gstart(w + 1, (par + 1) % NB)

                    sstart(w, par)
                return cc

            lax.fori_loop(0, cnt // NB, outer, 0)
            swait(cnt - 2, cnt % NB)
            swait(cnt - 1, (cnt + 1) % NB)

        off = 0
        for cnt in phases:
            run_phase(off, cnt)
            off += cnt
        plsc.subcore_barrier()

        def copy_out(off, nrows):
            pltpu.sync_copy(acc.at[pl.ds(off, nrows)],
                            out_hbm.at[pl.ds(c * N + off, nrows)])

        _per_tile_rows(s, copy_out)

    return pl.kernel(
        body,
        out_type=jax.ShapeDtypeStruct((2 * N, 128), jnp.float32),
        mesh=plsc.VectorSubcoreMesh(**_MESH),
        scratch_types=(
            [pltpu.VMEM((max((40, 40) if split else (24, 16)), CHUNK),
                        jnp.int32)] * 2
            + [pltpu.VMEM((CHUNK, 128), jnp.float32)] * NB
            + [pltpu.VMEM_SHARED((N_ACC, 128), jnp.float32)]
            + [pltpu.SemaphoreType.DMA] * (2 * NB)
        ),
    )


_prop256 = _make_prop(True)
_prop128p = _make_prop(False)


def _dinv_of(p_ref):
    return lax.rsqrt(p_ref[0, :, 0:1] + p_ref[1, :, 0:1])


def _mm1_body(x_ref, w_ref, p_ref, o_ref):
    dinv = _dinv_of(p_ref)
    v = jnp.dot(x_ref[...], w_ref[...], preferred_element_type=jnp.float32)
    v = v * dinv
    o_ref[0] = v[:, :128]
    o_ref[1] = v[:, 128:]


def _mm2_body(s1_ref, y1_ref, p_ref, b_ref, w_ref, o_ref):
    dinv = _dinv_of(p_ref)
    h = (jnp.concatenate([s1_ref[0] + y1_ref[0], s1_ref[1] + y1_ref[1]],
                         axis=1) * dinv + b_ref[...])
    h = jnp.maximum(h, 0.0)
    o_ref[...] = jnp.dot(h, w_ref[...],
                         preferred_element_type=jnp.float32) * dinv


def _z_body(s2_ref, y2_ref, p_ref, b_ref, z_ref, zs_ref):
    dinv = _dinv_of(p_ref)
    z = (s2_ref[0] + s2_ref[1] + y2_ref[...]) * dinv + b_ref[...]
    z_ref[...] = z
    zs_ref[...] = z * dinv


def _out_body(s3_ref, zs_ref, p_ref, w_ref, b_ref, o_ref):
    dinv = _dinv_of(p_ref)
    pz = (s3_ref[0] + s3_ref[1] + zs_ref[...]) * dinv
    o_ref[...] = (jnp.dot(pz, w_ref[...], preferred_element_type=jnp.float32)
                  + b_ref[...])


def _p_spec():
    return pl.BlockSpec((2, BM, 16), lambda i: (0, i, 0))


def _mm1(x, w1, p):
    return pl.pallas_call(
        _mm1_body,
        grid=(N // BM,),
        in_specs=[pl.BlockSpec((BM, 256), lambda i: (i, 0)),
                  pl.BlockSpec((256, 256), lambda i: (0, 0)),
                  _p_spec()],
        out_specs=pl.BlockSpec((2, BM, 128), lambda i: (0, i, 0)),
        out_shape=jax.ShapeDtypeStruct((2, N, 128), jnp.float32),
    )(x, w1, p)


def _mm2(s1, y1, p, b1, w2):
    return pl.pallas_call(
        _mm2_body,
        grid=(N // BM,),
        in_specs=[pl.BlockSpec((2, BM, 128), lambda i: (0, i, 0)),
                  pl.BlockSpec((2, BM, 128), lambda i: (0, i, 0)),
                  _p_spec(),
                  pl.BlockSpec((1, 256), lambda i: (0, 0)),
                  pl.BlockSpec((256, 128), lambda i: (0, 0))],
        out_specs=pl.BlockSpec((BM, 128), lambda i: (i, 0)),
        out_shape=jax.ShapeDtypeStruct((N, 128), jnp.float32),
    )(s1, y1, p, b1, w2)


def _zk(s2, y2, p, b2):
    return pl.pallas_call(
        _z_body,
        grid=(N // BM,),
        in_specs=[pl.BlockSpec((2, BM, 128), lambda i: (0, i, 0)),
                  pl.BlockSpec((BM, 128), lambda i: (i, 0)),
                  _p_spec(),
                  pl.BlockSpec((1, 128), lambda i: (0, 0))],
        out_specs=[pl.BlockSpec((BM, 128), lambda i: (i, 0)),
                   pl.BlockSpec((BM, 128), lambda i: (i, 0))],
        out_shape=[jax.ShapeDtypeStruct((N, 128), jnp.float32),
                   jax.ShapeDtypeStruct((N, 128), jnp.float32)],
    )(s2, y2, p, b2)


def _outk(s3, zs, p, w3, b3):
    return pl.pallas_call(
        _out_body,
        grid=(N // BM,),
        in_specs=[pl.BlockSpec((2, BM, 128), lambda i: (0, i, 0)),
                  pl.BlockSpec((BM, 128), lambda i: (i, 0)),
                  _p_spec(),
                  pl.BlockSpec((128, 256), lambda i: (0, 0)),
                  pl.BlockSpec((1, 256), lambda i: (0, 0))],
        out_specs=pl.BlockSpec((BM, 256), lambda i: (i, 0)),
        out_shape=jax.ShapeDtypeStruct((N, 256), jnp.float32),
    )(s3, zs, p, w3, b3)


def kernel(x, edge_index, W1, b1, W2, b2, W3, b3):
    src = edge_index[0].astype(jnp.int32)
    dst = edge_index[1].astype(jnp.int32)
    pad = E_PAD - E
    fill = jnp.arange(pad, dtype=jnp.int32)
    src_p = jnp.concatenate([src, fill % N])
    dst_p = jnp.concatenate([dst, N + fill % (N_ACC - N)])
    src2 = src_p.reshape(-1, CHUNK)
    srcB = jnp.concatenate([src_p, src_p + N]).reshape(-1, CHUNK)
    dst2 = dst_p.reshape(-1, CHUNK)
    vals = jnp.stack([jnp.ones((CHUNK, 16), jnp.float32),
                      jnp.zeros((CHUNK, 16), jnp.float32)])

    p = _deg(dst2, vals).reshape(2, N, 16)
    y1 = _mm1(x, W1, p)
    s1 = _prop256(y1.reshape(2 * N, 128), srcB, dst2).reshape(2, N, 128)
    y2 = _mm2(s1, y1, p, b1.reshape(1, -1), W2)
    s2 = _prop128p(y2, src2, dst2).reshape(2, N, 128)
    z, zs = _zk(s2, y2, p, b2.reshape(1, -1))
    s3 = _prop128p(zs, src2, dst2).reshape(2, N, 128)
    out = _outk(s3, zs, p, W3, b3.reshape(1, -1))
    return (out, z)


# speedup vs baseline: 17.2925x; 1.0180x over previous
import functools

import jax
import jax.numpy as jnp
from jax import lax
from jax.experimental import pallas as pl
from jax.experimental.pallas import tpu as pltpu
from jax.experimental.pallas import tpu_sc as plsc

N = 10000
E = 160000
E_PAD = 163840
CHUNK = 128
N_ACC = 10240
NT = 16
RPT = 632
BM = 400

_MESH = dict(core_axis_name="c", subcore_axis_name="s")


def _per_tile_rows(s, fn):
    @pl.when(s < NT - 1)
    def _():
        fn(s * RPT, RPT)

    @pl.when(s == NT - 1)
    def _():
        fn((NT - 1) * RPT, N - (NT - 1) * RPT)


def _deg_body(dst_hbm, vals_hbm, out_hbm, dstb, ones_v, acc, sem):
    c = lax.axis_index("c")
    s = lax.axis_index("s")
    nw = E_PAD // (2 * NT * CHUNK)
    pltpu.sync_copy(vals_hbm.at[c], ones_v)
    pltpu.sync_copy(dst_hbm.at[pl.ds((c * NT + s) * nw, nw)], dstb)
    for k in range(5):
        pltpu.sync_copy(ones_v, acc.at[pl.ds(s * 640 + k * CHUNK, CHUNK)])
    plsc.subcore_barrier()

    def fire(w, carry):
        pltpu.async_copy(ones_v, acc.at[dstb.at[w]], sem, add=True)
        return carry

    def drain(w, carry):
        pltpu.make_async_copy(ones_v, acc.at[dstb.at[w]], sem).wait()
        return carry

    lax.fori_loop(0, nw, fire, 0)
    lax.fori_loop(0, nw, drain, 0)
    plsc.subcore_barrier()

    def copy_out(off, nrows):
        pltpu.sync_copy(acc.at[pl.ds(off, nrows)],
                        out_hbm.at[pl.ds(c * N + off, nrows)])

    _per_tile_rows(s, copy_out)


@functools.partial(
    pl.kernel,
    out_type=jax.ShapeDtypeStruct((2 * N, 16), jnp.float32),
    mesh=plsc.VectorSubcoreMesh(**_MESH),
    scratch_types=[
        pltpu.VMEM((E_PAD // (2 * NT * CHUNK), CHUNK), jnp.int32),
        pltpu.VMEM((CHUNK, 16), jnp.float32),
        pltpu.VMEM_SHARED((10240, 16), jnp.float32),
        pltpu.SemaphoreType.DMA,
    ],
)
def _deg(dst_hbm, vals_hbm, out_hbm, dstb, ones_v, acc, sem):
    _deg_body(dst_hbm, vals_hbm, out_hbm, dstb, ones_v, acc, sem)


NB = 2


def _make_prop(split):
    nw = (E_PAD // (NT * CHUNK)) if split else (E_PAD // (2 * NT * CHUNK))

    phases = (40, 40) if split else (24, 16)
    nbuf = max(phases)

    def body(y_hbm, src_hbm, dst_hbm, out_hbm, srcb, dstb, r0, r1, acc,
             *sems):
        rows = (r0, r1)
        gsem = sems[:NB]
        ssem = sems[NB:]
        c = lax.axis_index("c")
        s = lax.axis_index("s")
        srow = (c * NT + s) * nw
        drow = s * nw if split else srow

        zbuf = rows[1]
        zsem = ssem[0]

        def zrow(j, cc):
            zbuf[j // 8, pl.ds((j % 8) * 16, 16)] = jnp.zeros(
                (16,), jnp.float32)
            return cc

        lax.fori_loop(0, CHUNK * 8, zrow, 0)

        def zero_in(off, nrows):
            nfull = nrows // CHUNK
            for k in range(nfull):
                pltpu.async_copy(zbuf, acc.at[pl.ds(off + k * CHUNK, CHUNK)],
                                 zsem)
            rem = nrows - nfull * CHUNK
            pltpu.async_copy(zbuf.at[pl.ds(0, rem)],
                             acc.at[pl.ds(off + nfull * CHUNK, rem)], zsem)

        def zero_drain(off, nrows):
            nfull = nrows // CHUNK
            for k in range(nfull):
                pltpu.make_async_copy(
                    zbuf, acc.at[pl.ds(off + k * CHUNK, CHUNK)],
                    zsem).wait()
            rem = nrows - nfull * CHUNK
            pltpu.make_async_copy(
                zbuf.at[pl.ds(0, rem)],
                acc.at[pl.ds(off + nfull * CHUNK, rem)], zsem).wait()

        _per_tile_rows(s, zero_in)

        def gstart(w, par):
            pltpu.async_copy(y_hbm.at[srcb.at[w]], rows[par], gsem[par])

        def gwait(w, par):
            pltpu.make_async_copy(y_hbm.at[srcb.at[w]], rows[par],
                                  gsem[par]).wait()

        def sstart(w, par):
            pltpu.async_copy(rows[par], acc.at[dstb.at[w]], ssem[par],
                             add=True)

        def swait(w, par):
            pltpu.make_async_copy(rows[par], acc.at[dstb.at[w]],
                                  ssem[par]).wait()

        def load_idx(off, cnt):
            pltpu.sync_copy(src_hbm.at[pl.ds(srow + off, cnt)],
                            srcb.at[pl.ds(0, cnt)])
            pltpu.sync_copy(dst_hbm.at[pl.ds(drow + off, cnt)],
                            dstb.at[pl.ds(0, cnt)])

        load_idx(0, phases[0])
        gstart(0, 0)
        _per_tile_rows(s, zero_drain)
        plsc.subcore_barrier()

        def run_phase(off, cnt, preloaded=False):
            if not preloaded:
                load_idx(off, cnt)
                gstart(0, 0)

            def outer(wo, cc):
                for par in range(NB):
                    w = wo * NB + par
                    gwait(w, par)

                    @pl.when(w + 1 < cnt)
                    def _():
                        @pl.when(w >= 1)
                        def _():
                            swait(w - 1, (par + 1) % NB)

                        gstart(w + 1, (par + 1) % NB)

                    sstart(w, par)
                return cc

            lax.fori_loop(0, cnt // NB, outer, 0)
            swait(cnt - 2, cnt % NB)
            swait(cnt - 1, (cnt + 1) % NB)

        off = 0
        for i, cnt in enumerate(phases):
            run_phase(off, cnt, preloaded=(i == 0))
            off += cnt
        plsc.subcore_barrier()

        def copy_out(off, nrows):
            pltpu.sync_copy(acc.at[pl.ds(off, nrows)],
                            out_hbm.at[pl.ds(c * N + off, nrows)])

        _per_tile_rows(s, copy_out)

    return pl.kernel(
        body,
        out_type=jax.ShapeDtypeStruct((2 * N, 128), jnp.float32),
        mesh=plsc.VectorSubcoreMesh(**_MESH),
        scratch_types=(
            [pltpu.VMEM((max((40, 40) if split else (24, 16)), CHUNK),
                        jnp.int32)] * 2
            + [pltpu.VMEM((CHUNK, 128), jnp.float32)] * NB
            + [pltpu.VMEM_SHARED((N_ACC, 128), jnp.float32)]
            + [pltpu.SemaphoreType.DMA] * (2 * NB)
        ),
    )


_prop256 = _make_prop(True)
_prop128p = _make_prop(False)


def _dinv_of(p_ref):
    return lax.rsqrt(p_ref[0, :, 0:1] + p_ref[1, :, 0:1])


def _mm1_body(x_ref, w_ref, o_ref):
    o_ref[...] = jnp.dot(x_ref[...], w_ref[...],
                         preferred_element_type=jnp.float32)


def _scale1_body(u_ref, p_ref, o_ref):
    dinv = _dinv_of(p_ref)
    v = u_ref[...] * dinv
    o_ref[0] = v[:, :128]
    o_ref[1] = v[:, 128:]


def _mm2_body(s1_ref, y1_ref, p_ref, b_ref, w_ref, o_ref):
    dinv = _dinv_of(p_ref)
    h = (jnp.concatenate([s1_ref[0] + y1_ref[0], s1_ref[1] + y1_ref[1]],
                         axis=1) * dinv + b_ref[...])
    h = jnp.maximum(h, 0.0)
    o_ref[...] = jnp.dot(h, w_ref[...],
                         preferred_element_type=jnp.float32) * dinv


def _z_body(s2_ref, y2_ref, p_ref, b_ref, z_ref, zs_ref):
    dinv = _dinv_of(p_ref)
    z = (s2_ref[0] + s2_ref[1] + y2_ref[...]) * dinv + b_ref[...]
    z_ref[...] = z
    zs_ref[...] = z * dinv


def _out_body(s3_ref, zs_ref, p_ref, w_ref, b_ref, o_ref):
    dinv = _dinv_of(p_ref)
    pz = (s3_ref[0] + s3_ref[1] + zs_ref[...]) * dinv
    o_ref[...] = (jnp.dot(pz, w_ref[...], preferred_element_type=jnp.float32)
                  + b_ref[...])


def _p_spec():
    return pl.BlockSpec((2, BM, 16), lambda i: (0, i, 0))


def _mm1(x, w1):
    return pl.pallas_call(
        _mm1_body,
        grid=(N // BM,),
        in_specs=[pl.BlockSpec((BM, 256), lambda i: (i, 0)),
                  pl.BlockSpec((256, 256), lambda i: (0, 0))],
        out_specs=pl.BlockSpec((BM, 256), lambda i: (i, 0)),
        out_shape=jax.ShapeDtypeStruct((N, 256), jnp.float32),
    )(x, w1)


def _scale1(u, p):
    return pl.pallas_call(
        _scale1_body,
        grid=(N // BM,),
        in_specs=[pl.BlockSpec((BM, 256), lambda i: (i, 0)),
                  _p_spec()],
        out_specs=pl.BlockSpec((2, BM, 128), lambda i: (0, i, 0)),
        out_shape=jax.ShapeDtypeStruct((2, N, 128), jnp.float32),
    )(u, p)


def _mm2(s1, y1, p, b1, w2):
    return pl.pallas_call(
        _mm2_body,
        grid=(N // BM,),
        in_specs=[pl.BlockSpec((2, BM, 128), lambda i: (0, i, 0)),
                  pl.BlockSpec((2, BM, 128), lambda i: (0, i, 0)),
                  _p_spec(),
                  pl.BlockSpec((1, 256), lambda i: (0, 0)),
                  pl.BlockSpec((256, 128), lambda i: (0, 0))],
        out_specs=pl.BlockSpec((BM, 128), lambda i: (i, 0)),
        out_shape=jax.ShapeDtypeStruct((N, 128), jnp.float32),
    )(s1, y1, p, b1, w2)


def _zk(s2, y2, p, b2):
    return pl.pallas_call(
        _z_body,
        grid=(N // BM,),
        in_specs=[pl.BlockSpec((2, BM, 128), lambda i: (0, i, 0)),
                  pl.BlockSpec((BM, 128), lambda i: (i, 0)),
                  _p_spec(),
                  pl.BlockSpec((1, 128), lambda i: (0, 0))],
        out_specs=[pl.BlockSpec((BM, 128), lambda i: (i, 0)),
                   pl.BlockSpec((BM, 128), lambda i: (i, 0))],
        out_shape=[jax.ShapeDtypeStruct((N, 128), jnp.float32),
                   jax.ShapeDtypeStruct((N, 128), jnp.float32)],
    )(s2, y2, p, b2)


def _outk(s3, zs, p, w3, b3):
    return pl.pallas_call(
        _out_body,
        grid=(N // BM,),
        in_specs=[pl.BlockSpec((2, BM, 128), lambda i: (0, i, 0)),
                  pl.BlockSpec((BM, 128), lambda i: (i, 0)),
                  _p_spec(),
                  pl.BlockSpec((128, 256), lambda i: (0, 0)),
                  pl.BlockSpec((1, 256), lambda i: (0, 0))],
        out_specs=pl.BlockSpec((BM, 256), lambda i: (i, 0)),
        out_shape=jax.ShapeDtypeStruct((N, 256), jnp.float32),
    )(s3, zs, p, w3, b3)


def kernel(x, edge_index, W1, b1, W2, b2, W3, b3):
    src = edge_index[0].astype(jnp.int32)
    dst = edge_index[1].astype(jnp.int32)
    pad = E_PAD - E
    fill = jnp.arange(pad, dtype=jnp.int32)
    src_p = jnp.concatenate([src, fill % N])
    dst_p = jnp.concatenate([dst, N + fill % (N_ACC - N)])
    src2 = src_p.reshape(-1, CHUNK)
    srcB = jnp.concatenate([src_p, src_p + N]).reshape(-1, CHUNK)
    dst2 = dst_p.reshape(-1, CHUNK)
    vals = jnp.stack([jnp.ones((CHUNK, 16), jnp.float32),
                      jnp.zeros((CHUNK, 16), jnp.float32)])

    u1 = _mm1(x, W1)
    p = _deg(dst2, vals).reshape(2, N, 16)
    y1 = _scale1(u1, p)
    s1 = _prop256(y1.reshape(2 * N, 128), srcB, dst2).reshape(2, N, 128)
    y2 = _mm2(s1, y1, p, b1.reshape(1, -1), W2)
    s2 = _prop128p(y2, src2, dst2).reshape(2, N, 128)
    z, zs = _zk(s2, y2, p, b2.reshape(1, -1))
    s3 = _prop128p(zs, src2, dst2).reshape(2, N, 128)
    out = _outk(s3, zs, p, W3, b3.reshape(1, -1))
    return (out, z)


# speedup vs baseline: 18.9444x; 1.0955x over previous
import functools

import jax
import jax.numpy as jnp
from jax import lax
from jax.experimental import pallas as pl
from jax.experimental.pallas import tpu as pltpu
from jax.experimental.pallas import tpu_sc as plsc

N = 10000
E = 160000
E_PAD = 163840
CHUNK = 128
N_ACC = 10240
NT = 16
RPT = 632
BM = 400

_MESH = dict(core_axis_name="c", subcore_axis_name="s")


def _per_tile_rows(s, fn):
    @pl.when(s < NT - 1)
    def _():
        fn(s * RPT, RPT)

    @pl.when(s == NT - 1)
    def _():
        fn((NT - 1) * RPT, N - (NT - 1) * RPT)


def _deg_body(dst_hbm, vals_hbm, out_hbm, dstb, ones_v, acc, sem):
    c = lax.axis_index("c")
    s = lax.axis_index("s")
    nw = E_PAD // (2 * NT * CHUNK)
    pltpu.sync_copy(vals_hbm.at[c], ones_v)
    pltpu.sync_copy(dst_hbm.at[pl.ds((c * NT + s) * nw, nw)], dstb)
    for k in range(5):
        pltpu.sync_copy(ones_v, acc.at[pl.ds(s * 640 + k * CHUNK, CHUNK)])
    plsc.subcore_barrier()

    def fire(w, carry):
        pltpu.async_copy(ones_v, acc.at[dstb.at[w]], sem, add=True)
        return carry

    def drain(w, carry):
        pltpu.make_async_copy(ones_v, acc.at[dstb.at[w]], sem).wait()
        return carry

    lax.fori_loop(0, nw, fire, 0)
    lax.fori_loop(0, nw, drain, 0)
    plsc.subcore_barrier()

    def copy_out(off, nrows):
        pltpu.sync_copy(acc.at[pl.ds(off, nrows)],
                        out_hbm.at[pl.ds(c * N + off, nrows)])

    _per_tile_rows(s, copy_out)


@functools.partial(
    pl.kernel,
    out_type=jax.ShapeDtypeStruct((2 * N, 16), jnp.float32),
    mesh=plsc.VectorSubcoreMesh(**_MESH),
    scratch_types=[
        pltpu.VMEM((E_PAD // (2 * NT * CHUNK), CHUNK), jnp.int32),
        pltpu.VMEM((CHUNK, 16), jnp.float32),
        pltpu.VMEM_SHARED((10240, 16), jnp.float32),
        pltpu.SemaphoreType.DMA,
    ],
)
def _deg(dst_hbm, vals_hbm, out_hbm, dstb, ones_v, acc, sem):
    _deg_body(dst_hbm, vals_hbm, out_hbm, dstb, ones_v, acc, sem)


NB = 2


def _make_prop(split):
    nw = (E_PAD // (NT * CHUNK)) if split else (E_PAD // (2 * NT * CHUNK))

    phases = (40, 40) if split else (24, 16)
    nbuf = max(phases)

    def body(y_hbm, src_hbm, dst_hbm, out_hbm, srcb, dstb, r0, r1, acc,
             *sems):
        rows = (r0, r1)
        gsem = sems[:NB]
        ssem = sems[NB:]
        c = lax.axis_index("c")
        s = lax.axis_index("s")
        srow = (c * NT + s) * nw
        drow = s * nw if split else srow

        zbuf = rows[1]
        zsem = ssem[0]

        def zrow(j, cc):
            zbuf[j // 8, pl.ds((j % 8) * 16, 16)] = jnp.zeros(
                (16,), jnp.float32)
            return cc

        lax.fori_loop(0, CHUNK * 8, zrow, 0)

        def zero_in(off, nrows):
            nfull = nrows // CHUNK
            for k in range(nfull):
                pltpu.async_copy(zbuf, acc.at[pl.ds(off + k * CHUNK, CHUNK)],
                                 zsem)
            rem = nrows - nfull * CHUNK
            pltpu.async_copy(zbuf.at[pl.ds(0, rem)],
                             acc.at[pl.ds(off + nfull * CHUNK, rem)], zsem)

        def zero_drain(off, nrows):
            nfull = nrows // CHUNK
            for k in range(nfull):
                pltpu.make_async_copy(
                    zbuf, acc.at[pl.ds(off + k * CHUNK, CHUNK)],
                    zsem).wait()
            rem = nrows - nfull * CHUNK
            pltpu.make_async_copy(
                zbuf.at[pl.ds(0, rem)],
                acc.at[pl.ds(off + nfull * CHUNK, rem)], zsem).wait()

        _per_tile_rows(s, zero_in)

        def gstart(w, par):
            pltpu.async_copy(y_hbm.at[srcb.at[w]], rows[par], gsem[par])

        def gwait(w, par):
            pltpu.make_async_copy(y_hbm.at[srcb.at[w]], rows[par],
                                  gsem[par]).wait()

        def sstart(w, par):
            pltpu.async_copy(rows[par], acc.at[dstb.at[w]], ssem[par],
                             add=True)

        def swait(w, par):
            pltpu.make_async_copy(rows[par], acc.at[dstb.at[w]],
                                  ssem[par]).wait()

        def load_idx(off, cnt):
            pltpu.sync_copy(src_hbm.at[pl.ds(srow + off, cnt)],
                            srcb.at[pl.ds(0, cnt)])
            pltpu.sync_copy(dst_hbm.at[pl.ds(drow + off, cnt)],
                            dstb.at[pl.ds(0, cnt)])

        load_idx(0, phases[0])
        gstart(0, 0)
        _per_tile_rows(s, zero_drain)
        plsc.subcore_barrier()

        def run_phase(off, cnt, preloaded=False):
            if not preloaded:
                load_idx(off, cnt)
                gstart(0, 0)

            def outer(wo, cc):
                for par in range(NB):
                    w = wo * NB + par

                    @pl.when(w >= 1)
                    def _():
                        swait(w - 1, (par + 1) % NB)

                    @pl.when(w + 1 < cnt)
                    def _():
                        gstart(w + 1, (par + 1) % NB)

                    gwait(w, par)
                    sstart(w, par)
                return cc

            lax.fori_loop(0, cnt // NB, outer, 0)
            swait(cnt - 1, (cnt + 1) % NB)

        off = 0
        for i, cnt in enumerate(phases):
            run_phase(off, cnt, preloaded=(i == 0))
            off += cnt
        plsc.subcore_barrier()

        def copy_out(off, nrows):
            pltpu.sync_copy(acc.at[pl.ds(off, nrows)],
                            out_hbm.at[pl.ds(c * N + off, nrows)])

        _per_tile_rows(s, copy_out)

    return pl.kernel(
        body,
        out_type=jax.ShapeDtypeStruct((2 * N, 128), jnp.float32),
        mesh=plsc.VectorSubcoreMesh(**_MESH),
        scratch_types=(
            [pltpu.VMEM((max((40, 40) if split else (24, 16)), CHUNK),
                        jnp.int32)] * 2
            + [pltpu.VMEM((CHUNK, 128), jnp.float32)] * NB
            + [pltpu.VMEM_SHARED((N_ACC, 128), jnp.float32)]
            + [pltpu.SemaphoreType.DMA] * (2 * NB)
        ),
    )


_prop256 = _make_prop(True)
_prop128p = _make_prop(False)


def _dinv_of(p_ref):
    return lax.rsqrt(p_ref[0, :, 0:1] + p_ref[1, :, 0:1])


def _mm1_body(x_ref, w_ref, o_ref):
    o_ref[...] = jnp.dot(x_ref[...], w_ref[...],
                         preferred_element_type=jnp.float32)


def _scale1_body(u_ref, p_ref, o_ref):
    dinv = _dinv_of(p_ref)
    v = u_ref[...] * dinv
    o_ref[0] = v[:, :128]
    o_ref[1] = v[:, 128:]


def _mm2_body(s1_ref, y1_ref, p_ref, b_ref, w_ref, o_ref):
    dinv = _dinv_of(p_ref)
    h = (jnp.concatenate([s1_ref[0] + y1_ref[0], s1_ref[1] + y1_ref[1]],
                         axis=1) * dinv + b_ref[...])
    h = jnp.maximum(h, 0.0)
    o_ref[...] = jnp.dot(h, w_ref[...],
                         preferred_element_type=jnp.float32) * dinv


def _z_body(s2_ref, y2_ref, p_ref, b_ref, z_ref, zs_ref):
    dinv = _dinv_of(p_ref)
    z = (s2_ref[0] + s2_ref[1] + y2_ref[...]) * dinv + b_ref[...]
    z_ref[...] = z
    zs_ref[...] = z * dinv


def _out_body(s3_ref, zs_ref, p_ref, w_ref, b_ref, o_ref):
    dinv = _dinv_of(p_ref)
    pz = (s3_ref[0] + s3_ref[1] + zs_ref[...]) * dinv
    o_ref[...] = (jnp.dot(pz, w_ref[...], preferred_element_type=jnp.float32)
                  + b_ref[...])


def _p_spec():
    return pl.BlockSpec((2, BM, 16), lambda i: (0, i, 0))


def _mm1(x, w1):
    return pl.pallas_call(
        _mm1_body,
        grid=(N // BM,),
        in_specs=[pl.BlockSpec((BM, 256), lambda i: (i, 0)),
                  pl.BlockSpec((256, 256), lambda i: (0, 0))],
        out_specs=pl.BlockSpec((BM, 256), lambda i: (i, 0)),
        out_shape=jax.ShapeDtypeStruct((N, 256), jnp.float32),
    )(x, w1)


def _scale1(u, p):
    return pl.pallas_call(
        _scale1_body,
        grid=(N // BM,),
        in_specs=[pl.BlockSpec((BM, 256), lambda i: (i, 0)),
                  _p_spec()],
        out_specs=pl.BlockSpec((2, BM, 128), lambda i: (0, i, 0)),
        out_shape=jax.ShapeDtypeStruct((2, N, 128), jnp.float32),
    )(u, p)


def _mm2(s1, y1, p, b1, w2):
    return pl.pallas_call(
        _mm2_body,
        grid=(N // BM,),
        in_specs=[pl.BlockSpec((2, BM, 128), lambda i: (0, i, 0)),
                  pl.BlockSpec((2, BM, 128), lambda i: (0, i, 0)),
                  _p_spec(),
                  pl.BlockSpec((1, 256), lambda i: (0, 0)),
                  pl.BlockSpec((256, 128), lambda i: (0, 0))],
        out_specs=pl.BlockSpec((BM, 128), lambda i: (i, 0)),
        out_shape=jax.ShapeDtypeStruct((N, 128), jnp.float32),
    )(s1, y1, p, b1, w2)


def _zk(s2, y2, p, b2):
    return pl.pallas_call(
        _z_body,
        grid=(N // BM,),
        in_specs=[pl.BlockSpec((2, BM, 128), lambda i: (0, i, 0)),
                  pl.BlockSpec((BM, 128), lambda i: (i, 0)),
                  _p_spec(),
                  pl.BlockSpec((1, 128), lambda i: (0, 0))],
        out_specs=[pl.BlockSpec((BM, 128), lambda i: (i, 0)),
                   pl.BlockSpec((BM, 128), lambda i: (i, 0))],
        out_shape=[jax.ShapeDtypeStruct((N, 128), jnp.float32),
                   jax.ShapeDtypeStruct((N, 128), jnp.float32)],
    )(s2, y2, p, b2)


def _outk(s3, zs, p, w3, b3):
    return pl.pallas_call(
        _out_body,
        grid=(N // BM,),
        in_specs=[pl.BlockSpec((2, BM, 128), lambda i: (0, i, 0)),
                  pl.BlockSpec((BM, 128), lambda i: (i, 0)),
                  _p_spec(),
                  pl.BlockSpec((128, 256), lambda i: (0, 0)),
                  pl.BlockSpec((1, 256), lambda i: (0, 0))],
        out_specs=pl.BlockSpec((BM, 256), lambda i: (i, 0)),
        out_shape=jax.ShapeDtypeStruct((N, 256), jnp.float32),
    )(s3, zs, p, w3, b3)


def kernel(x, edge_index, W1, b1, W2, b2, W3, b3):
    src = edge_index[0].astype(jnp.int32)
    dst = edge_index[1].astype(jnp.int32)
    pad = E_PAD - E
    fill = jnp.arange(pad, dtype=jnp.int32)
    src_p = jnp.concatenate([src, fill % N])
    dst_p = jnp.concatenate([dst, N + fill % (N_ACC - N)])
    src2 = src_p.reshape(-1, CHUNK)
    srcB = jnp.concatenate([src_p, src_p + N]).reshape(-1, CHUNK)
    dst2 = dst_p.reshape(-1, CHUNK)
    vals = jnp.stack([jnp.ones((CHUNK, 16), jnp.float32),
                      jnp.zeros((CHUNK, 16), jnp.float32)])

    u1 = _mm1(x, W1)
    p = _deg(dst2, vals).reshape(2, N, 16)
    y1 = _scale1(u1, p)
    s1 = _prop256(y1.reshape(2 * N, 128), srcB, dst2).reshape(2, N, 128)
    y2 = _mm2(s1, y1, p, b1.reshape(1, -1), W2)
    s2 = _prop128p(y2, src2, dst2).reshape(2, N, 128)
    z, zs = _zk(s2, y2, p, b2.reshape(1, -1))
    s3 = _prop128p(zs, src2, dst2).reshape(2, N, 128)
    out = _outk(s3, zs, p, W3, b3.reshape(1, -1))
    return (out, z)


# speedup vs baseline: 21.0603x; 1.1117x over previous
import functools

import jax
import jax.numpy as jnp
from jax import lax
from jax.experimental import pallas as pl
from jax.experimental.pallas import tpu as pltpu
from jax.experimental.pallas import tpu_sc as plsc

N = 10000
E = 160000
E_PAD = 163840
CHUNK = 128
N_ACC = 10240
NT = 16
RPT = 632
BM = 1000

_MESH = dict(core_axis_name="c", subcore_axis_name="s")


def _per_tile_rows(s, fn):
    @pl.when(s < NT - 1)
    def _():
        fn(s * RPT, RPT)

    @pl.when(s == NT - 1)
    def _():
        fn((NT - 1) * RPT, N - (NT - 1) * RPT)


def _deg_body(dst_hbm, vals_hbm, out_hbm, dstb, ones_v, acc, sem):
    c = lax.axis_index("c")
    s = lax.axis_index("s")
    nw = E_PAD // (2 * NT * CHUNK)
    pltpu.sync_copy(vals_hbm.at[c], ones_v)
    pltpu.sync_copy(dst_hbm.at[pl.ds((c * NT + s) * nw, nw)], dstb)
    for k in range(5):
        pltpu.sync_copy(ones_v, acc.at[pl.ds(s * 640 + k * CHUNK, CHUNK)])
    plsc.subcore_barrier()

    def fire(w, carry):
        pltpu.async_copy(ones_v, acc.at[dstb.at[w]], sem, add=True)
        return carry

    def drain(w, carry):
        pltpu.make_async_copy(ones_v, acc.at[dstb.at[w]], sem).wait()
        return carry

    lax.fori_loop(0, nw, fire, 0)
    lax.fori_loop(0, nw, drain, 0)
    plsc.subcore_barrier()

    def copy_out(off, nrows):
        pltpu.sync_copy(acc.at[pl.ds(off, nrows)],
                        out_hbm.at[pl.ds(c * N + off, nrows)])

    _per_tile_rows(s, copy_out)


@functools.partial(
    pl.kernel,
    out_type=jax.ShapeDtypeStruct((2 * N, 16), jnp.float32),
    mesh=plsc.VectorSubcoreMesh(**_MESH),
    scratch_types=[
        pltpu.VMEM((E_PAD // (2 * NT * CHUNK), CHUNK), jnp.int32),
        pltpu.VMEM((CHUNK, 16), jnp.float32),
        pltpu.VMEM_SHARED((10240, 16), jnp.float32),
        pltpu.SemaphoreType.DMA,
    ],
)
def _deg(dst_hbm, vals_hbm, out_hbm, dstb, ones_v, acc, sem):
    _deg_body(dst_hbm, vals_hbm, out_hbm, dstb, ones_v, acc, sem)


NB = 2


def _make_prop(split):
    nw = (E_PAD // (NT * CHUNK)) if split else (E_PAD // (2 * NT * CHUNK))

    phases = (40, 40) if split else (24, 16)
    nbuf = max(phases)

    def body(y_hbm, src_hbm, dst_hbm, out_hbm, srcb, dstb, r0, r1, acc,
             *sems):
        rows = (r0, r1)
        gsem = sems[:NB]
        ssem = sems[NB:]
        c = lax.axis_index("c")
        s = lax.axis_index("s")
        srow = (c * NT + s) * nw
        drow = s * nw if split else srow

        zbuf = rows[1]
        zsem = ssem[0]

        def zrow(j, cc):
            zbuf[j // 8, pl.ds((j % 8) * 16, 16)] = jnp.zeros(
                (16,), jnp.float32)
            return cc

        lax.fori_loop(0, CHUNK * 8, zrow, 0)

        def zero_in(off, nrows):
            nfull = nrows // CHUNK
            for k in range(nfull):
                pltpu.async_copy(zbuf, acc.at[pl.ds(off + k * CHUNK, CHUNK)],
                                 zsem)
            rem = nrows - nfull * CHUNK
            pltpu.async_copy(zbuf.at[pl.ds(0, rem)],
                             acc.at[pl.ds(off + nfull * CHUNK, rem)], zsem)

        def zero_drain(off, nrows):
            nfull = nrows // CHUNK
            for k in range(nfull):
                pltpu.make_async_copy(
                    zbuf, acc.at[pl.ds(off + k * CHUNK, CHUNK)],
                    zsem).wait()
            rem = nrows - nfull * CHUNK
            pltpu.make_async_copy(
                zbuf.at[pl.ds(0, rem)],
                acc.at[pl.ds(off + nfull * CHUNK, rem)], zsem).wait()

        _per_tile_rows(s, zero_in)

        def gstart(w, par):
            pltpu.async_copy(y_hbm.at[srcb.at[w]], rows[par], gsem[par])

        def gwait(w, par):
            pltpu.make_async_copy(y_hbm.at[srcb.at[w]], rows[par],
                                  gsem[par]).wait()

        def sstart(w, par):
            pltpu.async_copy(rows[par], acc.at[dstb.at[w]], ssem[par],
                             add=True)

        def swait(w, par):
            pltpu.make_async_copy(rows[par], acc.at[dstb.at[w]],
                                  ssem[par]).wait()

        def load_idx(off, cnt):
            pltpu.sync_copy(src_hbm.at[pl.ds(srow + off, cnt)],
                            srcb.at[pl.ds(0, cnt)])
            pltpu.sync_copy(dst_hbm.at[pl.ds(drow + off, cnt)],
                            dstb.at[pl.ds(0, cnt)])

        load_idx(0, phases[0])
        gstart(0, 0)
        _per_tile_rows(s, zero_drain)
        plsc.subcore_barrier()

        def run_phase(off, cnt, preloaded=False):
            if not preloaded:
                load_idx(off, cnt)
                gstart(0, 0)

            def outer(wo, cc):
                for par in range(NB):
                    w = wo * NB + par

                    @pl.when(w >= 1)
                    def _():
                        swait(w - 1, (par + 1) % NB)

                    @pl.when(w + 1 < cnt)
                    def _():
                        gstart(w + 1, (par + 1) % NB)

                    gwait(w, par)
                    sstart(w, par)
                return cc

            lax.fori_loop(0, cnt // NB, outer, 0)
            swait(cnt - 1, (cnt + 1) % NB)

        off = 0
        for i, cnt in enumerate(phases):
            run_phase(off, cnt, preloaded=(i == 0))
            off += cnt
        plsc.subcore_barrier()

        def copy_out(off, nrows):
            pltpu.sync_copy(acc.at[pl.ds(off, nrows)],
                            out_hbm.at[pl.ds(c * N + off, nrows)])

        _per_tile_rows(s, copy_out)

    return pl.kernel(
        body,
        out_type=jax.ShapeDtypeStruct((2 * N, 128), jnp.float32),
        mesh=plsc.VectorSubcoreMesh(**_MESH),
        scratch_types=(
            [pltpu.VMEM((max((40, 40) if split else (24, 16)), CHUNK),
                        jnp.int32)] * 2
            + [pltpu.VMEM((CHUNK, 128), jnp.float32)] * NB
            + [pltpu.VMEM_SHARED((N_ACC, 128), jnp.float32)]
            + [pltpu.SemaphoreType.DMA] * (2 * NB)
        ),
    )


_prop256 = _make_prop(True)
_prop128p = _make_prop(False)


def _dinv_of(p_ref):
    return lax.rsqrt(p_ref[0, :, 0:1] + p_ref[1, :, 0:1])


def _mm1_body(x_ref, w_ref, o_ref):
    o_ref[...] = jnp.dot(x_ref[...], w_ref[...],
                         preferred_element_type=jnp.float32)


def _scale1_body(u_ref, p_ref, o_ref):
    dinv = _dinv_of(p_ref)
    v = u_ref[...] * dinv
    o_ref[0] = v[:, :128]
    o_ref[1] = v[:, 128:]


def _mm2_body(s1_ref, y1_ref, p_ref, b_ref, w_ref, o_ref):
    dinv = _dinv_of(p_ref)
    h = (jnp.concatenate([s1_ref[0] + y1_ref[0], s1_ref[1] + y1_ref[1]],
                         axis=1) * dinv + b_ref[...])
    h = jnp.maximum(h, 0.0)
    o_ref[...] = jnp.dot(h, w_ref[...],
                         preferred_element_type=jnp.float32) * dinv


def _z_body(s2_ref, y2_ref, p_ref, b_ref, z_ref, zs_ref):
    dinv = _dinv_of(p_ref)
    z = (s2_ref[0] + s2_ref[1] + y2_ref[...]) * dinv + b_ref[...]
    z_ref[...] = z
    zs_ref[...] = z * dinv


def _out_body(s3_ref, zs_ref, p_ref, w_ref, b_ref, o_ref):
    dinv = _dinv_of(p_ref)
    pz = (s3_ref[0] + s3_ref[1] + zs_ref[...]) * dinv
    o_ref[...] = (jnp.dot(pz, w_ref[...], preferred_element_type=jnp.float32)
                  + b_ref[...])


def _p_spec():
    return pl.BlockSpec((2, BM, 16), lambda i: (0, i, 0))


def _mm1(x, w1):
    return pl.pallas_call(
        _mm1_body,
        grid=(N // BM,),
        in_specs=[pl.BlockSpec((BM, 256), lambda i: (i, 0)),
                  pl.BlockSpec((256, 256), lambda i: (0, 0))],
        out_specs=pl.BlockSpec((BM, 256), lambda i: (i, 0)),
        out_shape=jax.ShapeDtypeStruct((N, 256), jnp.float32),
    )(x, w1)


def _scale1(u, p):
    return pl.pallas_call(
        _scale1_body,
        grid=(N // BM,),
        in_specs=[pl.BlockSpec((BM, 256), lambda i: (i, 0)),
                  _p_spec()],
        out_specs=pl.BlockSpec((2, BM, 128), lambda i: (0, i, 0)),
        out_shape=jax.ShapeDtypeStruct((2, N, 128), jnp.float32),
    )(u, p)


def _mm2(s1, y1, p, b1, w2):
    return pl.pallas_call(
        _mm2_body,
        grid=(N // BM,),
        in_specs=[pl.BlockSpec((2, BM, 128), lambda i: (0, i, 0)),
                  pl.BlockSpec((2, BM, 128), lambda i: (0, i, 0)),
                  _p_spec(),
                  pl.BlockSpec((1, 256), lambda i: (0, 0)),
                  pl.BlockSpec((256, 128), lambda i: (0, 0))],
        out_specs=pl.BlockSpec((BM, 128), lambda i: (i, 0)),
        out_shape=jax.ShapeDtypeStruct((N, 128), jnp.float32),
    )(s1, y1, p, b1, w2)


def _zk(s2, y2, p, b2):
    return pl.pallas_call(
        _z_body,
        grid=(N // BM,),
        in_specs=[pl.BlockSpec((2, BM, 128), lambda i: (0, i, 0)),
                  pl.BlockSpec((BM, 128), lambda i: (i, 0)),
                  _p_spec(),
                  pl.BlockSpec((1, 128), lambda i: (0, 0))],
        out_specs=[pl.BlockSpec((BM, 128), lambda i: (i, 0)),
                   pl.BlockSpec((BM, 128), lambda i: (i, 0))],
        out_shape=[jax.ShapeDtypeStruct((N, 128), jnp.float32),
                   jax.ShapeDtypeStruct((N, 128), jnp.float32)],
    )(s2, y2, p, b2)


def _outk(s3, zs, p, w3, b3):
    return pl.pallas_call(
        _out_body,
        grid=(N // BM,),
        in_specs=[pl.BlockSpec((2, BM, 128), lambda i: (0, i, 0)),
                  pl.BlockSpec((BM, 128), lambda i: (i, 0)),
                  _p_spec(),
                  pl.BlockSpec((128, 256), lambda i: (0, 0)),
                  pl.BlockSpec((1, 256), lambda i: (0, 0))],
        out_specs=pl.BlockSpec((BM, 256), lambda i: (i, 0)),
        out_shape=jax.ShapeDtypeStruct((N, 256), jnp.float32),
    )(s3, zs, p, w3, b3)


def kernel(x, edge_index, W1, b1, W2, b2, W3, b3):
    src = edge_index[0].astype(jnp.int32)
    dst = edge_index[1].astype(jnp.int32)
    pad = E_PAD - E
    fill = jnp.arange(pad, dtype=jnp.int32)
    src_p = jnp.concatenate([src, fill % N])
    dst_p = jnp.concatenate([dst, N + fill % (N_ACC - N)])
    src2 = src_p.reshape(-1, CHUNK)
    srcB = jnp.concatenate([src_p, src_p + N]).reshape(-1, CHUNK)
    dst2 = dst_p.reshape(-1, CHUNK)
    vals = jnp.stack([jnp.ones((CHUNK, 16), jnp.float32),
                      jnp.zeros((CHUNK, 16), jnp.float32)])

    u1 = _mm1(x, W1)
    p = _deg(dst2, vals).reshape(2, N, 16)
    y1 = _scale1(u1, p)
    s1 = _prop256(y1.reshape(2 * N, 128), srcB, dst2).reshape(2, N, 128)
    y2 = _mm2(s1, y1, p, b1.reshape(1, -1), W2)
    s2 = _prop128p(y2, src2, dst2).reshape(2, N, 128)
    z, zs = _zk(s2, y2, p, b2.reshape(1, -1))
    s3 = _prop128p(zs, src2, dst2).reshape(2, N, 128)
    out = _outk(s3, zs, p, W3, b3.reshape(1, -1))
    return (out, z)


# speedup vs baseline: 21.3946x; 1.0159x over previous
import functools

import jax
import jax.numpy as jnp
from jax import lax
from jax.experimental import pallas as pl
from jax.experimental.pallas import tpu as pltpu
from jax.experimental.pallas import tpu_sc as plsc

N = 10000
E = 160000
E_PAD = 163840
CHUNK = 128
N_ACC = 10240
NT = 16
RPT = 632
BM = 2000

_MESH = dict(core_axis_name="c", subcore_axis_name="s")


def _per_tile_rows(s, fn):
    @pl.when(s < NT - 1)
    def _():
        fn(s * RPT, RPT)

    @pl.when(s == NT - 1)
    def _():
        fn((NT - 1) * RPT, N - (NT - 1) * RPT)


def _deg_body(dst_hbm, vals_hbm, out_hbm, dstb, ones_v, acc, sem):
    c = lax.axis_index("c")
    s = lax.axis_index("s")
    nw = E_PAD // (2 * NT * CHUNK)
    pltpu.sync_copy(vals_hbm.at[c], ones_v)
    pltpu.sync_copy(dst_hbm.at[pl.ds((c * NT + s) * nw, nw)], dstb)
    for k in range(5):
        pltpu.sync_copy(ones_v, acc.at[pl.ds(s * 640 + k * CHUNK, CHUNK)])
    plsc.subcore_barrier()

    def fire(w, carry):
        pltpu.async_copy(ones_v, acc.at[dstb.at[w]], sem, add=True)
        return carry

    def drain(w, carry):
        pltpu.make_async_copy(ones_v, acc.at[dstb.at[w]], sem).wait()
        return carry

    lax.fori_loop(0, nw, fire, 0)
    lax.fori_loop(0, nw, drain, 0)
    plsc.subcore_barrier()

    def copy_out(off, nrows):
        pltpu.sync_copy(acc.at[pl.ds(off, nrows)],
                        out_hbm.at[pl.ds(c * N + off, nrows)])

    _per_tile_rows(s, copy_out)


@functools.partial(
    pl.kernel,
    out_type=jax.ShapeDtypeStruct((2 * N, 16), jnp.float32),
    mesh=plsc.VectorSubcoreMesh(**_MESH),
    scratch_types=[
        pltpu.VMEM((E_PAD // (2 * NT * CHUNK), CHUNK), jnp.int32),
        pltpu.VMEM((CHUNK, 16), jnp.float32),
        pltpu.VMEM_SHARED((10240, 16), jnp.float32),
        pltpu.SemaphoreType.DMA,
    ],
)
def _deg(dst_hbm, vals_hbm, out_hbm, dstb, ones_v, acc, sem):
    _deg_body(dst_hbm, vals_hbm, out_hbm, dstb, ones_v, acc, sem)


NB = 2


def _make_prop(split):
    nw = (E_PAD // (NT * CHUNK)) if split else (E_PAD // (2 * NT * CHUNK))

    phases = (40, 40) if split else (24, 16)
    nbuf = max(phases)

    def body(y_hbm, src_hbm, dst_hbm, out_hbm, srcb, dstb, r0, r1, acc,
             *sems):
        rows = (r0, r1)
        gsem = sems[:NB]
        ssem = sems[NB:]
        c = lax.axis_index("c")
        s = lax.axis_index("s")
        srow = (c * NT + s) * nw
        drow = s * nw if split else srow

        zbuf = rows[1]
        zsem = ssem[0]

        def zrow(j, cc):
            zbuf[j // 8, pl.ds((j % 8) * 16, 16)] = jnp.zeros(
                (16,), jnp.float32)
            return cc

        lax.fori_loop(0, CHUNK * 8, zrow, 0)

        def zero_in(off, nrows):
            nfull = nrows // CHUNK
            for k in range(nfull):
                pltpu.async_copy(zbuf, acc.at[pl.ds(off + k * CHUNK, CHUNK)],
                                 zsem)
            rem = nrows - nfull * CHUNK
            pltpu.async_copy(zbuf.at[pl.ds(0, rem)],
                             acc.at[pl.ds(off + nfull * CHUNK, rem)], zsem)

        def zero_drain(off, nrows):
            nfull = nrows // CHUNK
            for k in range(nfull):
                pltpu.make_async_copy(
                    zbuf, acc.at[pl.ds(off + k * CHUNK, CHUNK)],
                    zsem).wait()
            rem = nrows - nfull * CHUNK
            pltpu.make_async_copy(
                zbuf.at[pl.ds(0, rem)],
                acc.at[pl.ds(off + nfull * CHUNK, rem)], zsem).wait()

        _per_tile_rows(s, zero_in)

        def gstart(w, par):
            pltpu.async_copy(y_hbm.at[srcb.at[w]], rows[par], gsem[par])

        def gwait(w, par):
            pltpu.make_async_copy(y_hbm.at[srcb.at[w]], rows[par],
                                  gsem[par]).wait()

        def sstart(w, par):
            pltpu.async_copy(rows[par], acc.at[dstb.at[w]], ssem[par],
                             add=True)

        def swait(w, par):
            pltpu.make_async_copy(rows[par], acc.at[dstb.at[w]],
                                  ssem[par]).wait()

        def load_idx(off, cnt):
            pltpu.sync_copy(src_hbm.at[pl.ds(srow + off, cnt)],
                            srcb.at[pl.ds(0, cnt)])
            pltpu.sync_copy(dst_hbm.at[pl.ds(drow + off, cnt)],
                            dstb.at[pl.ds(0, cnt)])

        load_idx(0, phases[0])
        gstart(0, 0)
        _per_tile_rows(s, zero_drain)
        plsc.subcore_barrier()

        def run_phase(off, cnt, preloaded=False):
            if not preloaded:
                load_idx(off, cnt)
                gstart(0, 0)

            def outer(wo, cc):
                for par in range(NB):
                    w = wo * NB + par

                    @pl.when(w >= 1)
                    def _():
                        swait(w - 1, (par + 1) % NB)

                    @pl.when(w + 1 < cnt)
                    def _():
                        gstart(w + 1, (par + 1) % NB)

                    gwait(w, par)
                    sstart(w, par)
                return cc

            lax.fori_loop(0, cnt // NB, outer, 0)
            swait(cnt - 1, (cnt + 1) % NB)

        off = 0
        for i, cnt in enumerate(phases):
            run_phase(off, cnt, preloaded=(i == 0))
            off += cnt
        plsc.subcore_barrier()

        def copy_out(off, nrows):
            pltpu.sync_copy(acc.at[pl.ds(off, nrows)],
                            out_hbm.at[pl.ds(c * N + off, nrows)])

        _per_tile_rows(s, copy_out)

    return pl.kernel(
        body,
        out_type=jax.ShapeDtypeStruct((2 * N, 128), jnp.float32),
        mesh=plsc.VectorSubcoreMesh(**_MESH),
        scratch_types=(
            [pltpu.VMEM((max((40, 40) if split else (24, 16)), CHUNK),
                        jnp.int32)] * 2
            + [pltpu.VMEM((CHUNK, 128), jnp.float32)] * NB
            + [pltpu.VMEM_SHARED((N_ACC, 128), jnp.float32)]
            + [pltpu.SemaphoreType.DMA] * (2 * NB)
        ),
    )


_prop256 = _make_prop(True)
_prop128p = _make_prop(False)


def _dinv_of(p_ref):
    return lax.rsqrt(p_ref[0, :, 0:1] + p_ref[1, :, 0:1])


def _mm1_body(x_ref, w_ref, o_ref):
    o_ref[...] = jnp.dot(x_ref[...], w_ref[...],
                         preferred_element_type=jnp.float32)


def _scale1_body(u_ref, p_ref, o_ref):
    dinv = _dinv_of(p_ref)
    v = u_ref[...] * dinv
    o_ref[0] = v[:, :128]
    o_ref[1] = v[:, 128:]


def _mm2_body(s1_ref, y1_ref, p_ref, b_ref, w_ref, o_ref):
    dinv = _dinv_of(p_ref)
    h = (jnp.concatenate([s1_ref[0] + y1_ref[0], s1_ref[1] + y1_ref[1]],
                         axis=1) * dinv + b_ref[...])
    h = jnp.maximum(h, 0.0)
    o_ref[...] = jnp.dot(h, w_ref[...],
                         preferred_element_type=jnp.float32) * dinv


def _z_body(s2_ref, y2_ref, p_ref, b_ref, z_ref, zs_ref):
    dinv = _dinv_of(p_ref)
    z = (s2_ref[0] + s2_ref[1] + y2_ref[...]) * dinv + b_ref[...]
    z_ref[...] = z
    zs_ref[...] = z * dinv


def _out_body(s3_ref, zs_ref, p_ref, w_ref, b_ref, o_ref):
    dinv = _dinv_of(p_ref)
    pz = (s3_ref[0] + s3_ref[1] + zs_ref[...]) * dinv
    o_ref[...] = (jnp.dot(pz, w_ref[...], preferred_element_type=jnp.float32)
                  + b_ref[...])


def _p_spec():
    return pl.BlockSpec((2, BM, 16), lambda i: (0, i, 0))


def _mm1(x, w1):
    return pl.pallas_call(
        _mm1_body,
        grid=(N // BM,),
        in_specs=[pl.BlockSpec((BM, 256), lambda i: (i, 0)),
                  pl.BlockSpec((256, 256), lambda i: (0, 0))],
        out_specs=pl.BlockSpec((BM, 256), lambda i: (i, 0)),
        out_shape=jax.ShapeDtypeStruct((N, 256), jnp.float32),
    )(x, w1)


def _scale1(u, p):
    return pl.pallas_call(
        _scale1_body,
        grid=(N // BM,),
        in_specs=[pl.BlockSpec((BM, 256), lambda i: (i, 0)),
                  _p_spec()],
        out_specs=pl.BlockSpec((2, BM, 128), lambda i: (0, i, 0)),
        out_shape=jax.ShapeDtypeStruct((2, N, 128), jnp.float32),
    )(u, p)


def _mm2(s1, y1, p, b1, w2):
    return pl.pallas_call(
        _mm2_body,
        grid=(N // BM,),
        in_specs=[pl.BlockSpec((2, BM, 128), lambda i: (0, i, 0)),
                  pl.BlockSpec((2, BM, 128), lambda i: (0, i, 0)),
                  _p_spec(),
                  pl.BlockSpec((1, 256), lambda i: (0, 0)),
                  pl.BlockSpec((256, 128), lambda i: (0, 0))],
        out_specs=pl.BlockSpec((BM, 128), lambda i: (i, 0)),
        out_shape=jax.ShapeDtypeStruct((N, 128), jnp.float32),
    )(s1, y1, p, b1, w2)


def _zk(s2, y2, p, b2):
    return pl.pallas_call(
        _z_body,
        grid=(N // BM,),
        in_specs=[pl.BlockSpec((2, BM, 128), lambda i: (0, i, 0)),
                  pl.BlockSpec((BM, 128), lambda i: (i, 0)),
                  _p_spec(),
                  pl.BlockSpec((1, 128), lambda i: (0, 0))],
        out_specs=[pl.BlockSpec((BM, 128), lambda i: (i, 0)),
                   pl.BlockSpec((BM, 128), lambda i: (i, 0))],
        out_shape=[jax.ShapeDtypeStruct((N, 128), jnp.float32),
                   jax.ShapeDtypeStruct((N, 128), jnp.float32)],
    )(s2, y2, p, b2)


def _outk(s3, zs, p, w3, b3):
    return pl.pallas_call(
        _out_body,
        grid=(N // BM,),
        in_specs=[pl.BlockSpec((2, BM, 128), lambda i: (0, i, 0)),
                  pl.BlockSpec((BM, 128), lambda i: (i, 0)),
                  _p_spec(),
                  pl.BlockSpec((128, 256), lambda i: (0, 0)),
                  pl.BlockSpec((1, 256), lambda i: (0, 0))],
        out_specs=pl.BlockSpec((BM, 256), lambda i: (i, 0)),
        out_shape=jax.ShapeDtypeStruct((N, 256), jnp.float32),
    )(s3, zs, p, w3, b3)


def kernel(x, edge_index, W1, b1, W2, b2, W3, b3):
    src = edge_index[0].astype(jnp.int32)
    dst = edge_index[1].astype(jnp.int32)
    pad = E_PAD - E
    fill = jnp.arange(pad, dtype=jnp.int32)
    src_p = jnp.concatenate([src, fill % N])
    dst_p = jnp.concatenate([dst, N + fill % (N_ACC - N)])
    src2 = src_p.reshape(-1, CHUNK)
    srcB = jnp.concatenate([src_p, src_p + N]).reshape(-1, CHUNK)
    dst2 = dst_p.reshape(-1, CHUNK)
    vals = jnp.stack([jnp.ones((CHUNK, 16), jnp.float32),
                      jnp.zeros((CHUNK, 16), jnp.float32)])

    u1 = _mm1(x, W1)
    p = _deg(dst2, vals).reshape(2, N, 16)
    y1 = _scale1(u1, p)
    s1 = _prop256(y1.reshape(2 * N, 128), srcB, dst2).reshape(2, N, 128)
    y2 = _mm2(s1, y1, p, b1.reshape(1, -1), W2)
    s2 = _prop128p(y2, src2, dst2).reshape(2, N, 128)
    z, zs = _zk(s2, y2, p, b2.reshape(1, -1))
    s3 = _prop128p(zs, src2, dst2).reshape(2, N, 128)
    out = _outk(s3, zs, p, W3, b3.reshape(1, -1))
    return (out, z)


# speedup vs baseline: 21.5247x; 1.0061x over previous
import functools

import jax
import jax.numpy as jnp
from jax import lax
from jax.experimental import pallas as pl
from jax.experimental.pallas import tpu as pltpu
from jax.experimental.pallas import tpu_sc as plsc

N = 10000
E = 160000
E_PAD = 163840
CHUNK = 128
N_ACC = 10240
NT = 16
RPT = 632
BM = 2000

_MESH = dict(core_axis_name="c", subcore_axis_name="s")


def _per_tile_rows(s, fn):
    @pl.when(s < NT - 1)
    def _():
        fn(s * RPT, RPT)

    @pl.when(s == NT - 1)
    def _():
        fn((NT - 1) * RPT, N - (NT - 1) * RPT)


def _deg_body(dst_hbm, vals_hbm, out_hbm, dstb, ones_v, acc, sem):
    c = lax.axis_index("c")
    s = lax.axis_index("s")
    nw = E_PAD // (2 * NT * CHUNK)
    pltpu.sync_copy(vals_hbm.at[c], ones_v)
    pltpu.sync_copy(dst_hbm.at[pl.ds((c * NT + s) * nw, nw)], dstb)
    for k in range(5):
        pltpu.sync_copy(ones_v, acc.at[pl.ds(s * 640 + k * CHUNK, CHUNK)])
    plsc.subcore_barrier()

    def fire(w, carry):
        pltpu.async_copy(ones_v, acc.at[dstb.at[w]], sem, add=True)
        return carry

    def drain(w, carry):
        pltpu.make_async_copy(ones_v, acc.at[dstb.at[w]], sem).wait()
        return carry

    lax.fori_loop(0, nw, fire, 0)
    lax.fori_loop(0, nw, drain, 0)
    plsc.subcore_barrier()

    def copy_out(off, nrows):
        pltpu.sync_copy(acc.at[pl.ds(off, nrows)],
                        out_hbm.at[pl.ds(c * N + off, nrows)])

    _per_tile_rows(s, copy_out)


@functools.partial(
    pl.kernel,
    out_type=jax.ShapeDtypeStruct((2 * N, 16), jnp.float32),
    mesh=plsc.VectorSubcoreMesh(**_MESH),
    scratch_types=[
        pltpu.VMEM((E_PAD // (2 * NT * CHUNK), CHUNK), jnp.int32),
        pltpu.VMEM((CHUNK, 16), jnp.float32),
        pltpu.VMEM_SHARED((10240, 16), jnp.float32),
        pltpu.SemaphoreType.DMA,
    ],
)
def _deg(dst_hbm, vals_hbm, out_hbm, dstb, ones_v, acc, sem):
    _deg_body(dst_hbm, vals_hbm, out_hbm, dstb, ones_v, acc, sem)


NB = 2


def _make_prop(split):
    nw = (E_PAD // (NT * CHUNK)) if split else (E_PAD // (2 * NT * CHUNK))

    phases = (40, 40) if split else (24, 16)
    nbuf = max(phases)

    def body(y_hbm, src_hbm, dst_hbm, out_hbm, srcb, dstb, r0, r1, acc,
             *sems):
        rows = (r0, r1)
        gsem = sems[:NB]
        ssem = sems[NB:]
        c = lax.axis_index("c")
        s = lax.axis_index("s")
        srow = (c * NT + s) * nw
        drow = s * nw if split else srow

        zbuf = rows[1]
        zsem = ssem[0]

        def zrow(j, cc):
            zbuf[j // 8, pl.ds((j % 8) * 16, 16)] = jnp.zeros(
                (16,), jnp.float32)
            return cc

        lax.fori_loop(0, CHUNK * 8, zrow, 0)

        def zero_in(off, nrows):
            nfull = nrows // CHUNK
            for k in range(nfull):
                pltpu.async_copy(zbuf, acc.at[pl.ds(off + k * CHUNK, CHUNK)],
                                 zsem)
            rem = nrows - nfull * CHUNK
            pltpu.async_copy(zbuf.at[pl.ds(0, rem)],
                             acc.at[pl.ds(off + nfull * CHUNK, rem)], zsem)

        def zero_drain(off, nrows):
            nfull = nrows // CHUNK
            for k in range(nfull):
                pltpu.make_async_copy(
                    zbuf, acc.at[pl.ds(off + k * CHUNK, CHUNK)],
                    zsem).wait()
            rem = nrows - nfull * CHUNK
            pltpu.make_async_copy(
                zbuf.at[pl.ds(0, rem)],
                acc.at[pl.ds(off + nfull * CHUNK, rem)], zsem).wait()

        _per_tile_rows(s, zero_in)

        def gstart(w, par):
            pltpu.async_copy(y_hbm.at[srcb.at[w]], rows[par], gsem[par])

        def gwait(w, par):
            pltpu.make_async_copy(y_hbm.at[srcb.at[w]], rows[par],
                                  gsem[par]).wait()

        def sstart(w, par):
            pltpu.async_copy(rows[par], acc.at[dstb.at[w]], ssem[par],
                             add=True)

        def swait(w, par):
            pltpu.make_async_copy(rows[par], acc.at[dstb.at[w]],
                                  ssem[par]).wait()

        def load_idx(off, cnt):
            pltpu.sync_copy(src_hbm.at[pl.ds(srow + off, cnt)],
                            srcb.at[pl.ds(0, cnt)])
            pltpu.sync_copy(dst_hbm.at[pl.ds(drow + off, cnt)],
                            dstb.at[pl.ds(0, cnt)])

        load_idx(0, phases[0])
        gstart(0, 0)
        _per_tile_rows(s, zero_drain)
        plsc.subcore_barrier()

        def run_phase(off, cnt, preloaded=False):
            if not preloaded:
                load_idx(off, cnt)
                gstart(0, 0)

            def outer(wo, cc):
                for par in range(NB):
                    w = wo * NB + par

                    @pl.when(w >= 1)
                    def _():
                        swait(w - 1, (par + 1) % NB)

                    @pl.when(w + 1 < cnt)
                    def _():
                        gstart(w + 1, (par + 1) % NB)

                    gwait(w, par)
                    sstart(w, par)
                return cc

            lax.fori_loop(0, cnt // NB, outer, 0)
            swait(cnt - 1, (cnt + 1) % NB)

        off = 0
        for i, cnt in enumerate(phases):
            run_phase(off, cnt, preloaded=(i == 0))
            off += cnt
        plsc.subcore_barrier()

        def copy_out(off, nrows):
            pltpu.sync_copy(acc.at[pl.ds(off, nrows)],
                            out_hbm.at[pl.ds(c * N + off, nrows)])

        _per_tile_rows(s, copy_out)

    return pl.kernel(
        body,
        out_type=jax.ShapeDtypeStruct((2 * N, 128), jnp.float32),
        mesh=plsc.VectorSubcoreMesh(**_MESH),
        scratch_types=(
            [pltpu.VMEM((max((40, 40) if split else (24, 16)), CHUNK),
                        jnp.int32)] * 2
            + [pltpu.VMEM((CHUNK, 128), jnp.float32)] * NB
            + [pltpu.VMEM_SHARED((N_ACC, 128), jnp.float32)]
            + [pltpu.SemaphoreType.DMA] * (2 * NB)
        ),
    )


_prop256 = _make_prop(True)
_prop128p = _make_prop(False)


def _dinv_of(p_ref):
    return lax.rsqrt(p_ref[0, :, 0:1] + p_ref[1, :, 0:1])


def _mm1_body(x_ref, w_ref, p_ref, o_ref):
    dinv = _dinv_of(p_ref)
    v = jnp.dot(x_ref[...], w_ref[...],
                preferred_element_type=jnp.float32) * dinv
    o_ref[0] = v[:, :128]
    o_ref[1] = v[:, 128:]


def _mm2_body(s1_ref, y1_ref, p_ref, b_ref, w_ref, o_ref):
    dinv = _dinv_of(p_ref)
    h = (jnp.concatenate([s1_ref[0] + y1_ref[0], s1_ref[1] + y1_ref[1]],
                         axis=1) * dinv + b_ref[...])
    h = jnp.maximum(h, 0.0)
    o_ref[...] = jnp.dot(h, w_ref[...],
                         preferred_element_type=jnp.float32) * dinv


def _z_body(s2_ref, y2_ref, p_ref, b_ref, z_ref, zs_ref):
    dinv = _dinv_of(p_ref)
    z = (s2_ref[0] + s2_ref[1] + y2_ref[...]) * dinv + b_ref[...]
    z_ref[...] = z
    zs_ref[...] = z * dinv


def _out_body(s3_ref, zs_ref, p_ref, w_ref, b_ref, o_ref):
    dinv = _dinv_of(p_ref)
    pz = (s3_ref[0] + s3_ref[1] + zs_ref[...]) * dinv
    o_ref[...] = (jnp.dot(pz, w_ref[...], preferred_element_type=jnp.float32)
                  + b_ref[...])


def _p_spec():
    return pl.BlockSpec((2, BM, 16), lambda i: (0, i, 0))


def _mm1(x, w1, p):
    return pl.pallas_call(
        _mm1_body,
        grid=(N // BM,),
        in_specs=[pl.BlockSpec((BM, 256), lambda i: (i, 0)),
                  pl.BlockSpec((256, 256), lambda i: (0, 0)),
                  _p_spec()],
        out_specs=pl.BlockSpec((2, BM, 128), lambda i: (0, i, 0)),
        out_shape=jax.ShapeDtypeStruct((2, N, 128), jnp.float32),
    )(x, w1, p)


def _mm2(s1, y1, p, b1, w2):
    return pl.pallas_call(
        _mm2_body,
        grid=(N // BM,),
        in_specs=[pl.BlockSpec((2, BM, 128), lambda i: (0, i, 0)),
                  pl.BlockSpec((2, BM, 128), lambda i: (0, i, 0)),
                  _p_spec(),
                  pl.BlockSpec((1, 256), lambda i: (0, 0)),
                  pl.BlockSpec((256, 128), lambda i: (0, 0))],
        out_specs=pl.BlockSpec((BM, 128), lambda i: (i, 0)),
        out_shape=jax.ShapeDtypeStruct((N, 128), jnp.float32),
    )(s1, y1, p, b1, w2)


def _zk(s2, y2, p, b2):
    return pl.pallas_call(
        _z_body,
        grid=(N // BM,),
        in_specs=[pl.BlockSpec((2, BM, 128), lambda i: (0, i, 0)),
                  pl.BlockSpec((BM, 128), lambda i: (i, 0)),
                  _p_spec(),
                  pl.BlockSpec((1, 128), lambda i: (0, 0))],
        out_specs=[pl.BlockSpec((BM, 128), lambda i: (i, 0)),
                   pl.BlockSpec((BM, 128), lambda i: (i, 0))],
        out_shape=[jax.ShapeDtypeStruct((N, 128), jnp.float32),
                   jax.ShapeDtypeStruct((N, 128), jnp.float32)],
    )(s2, y2, p, b2)


def _outk(s3, zs, p, w3, b3):
    return pl.pallas_call(
        _out_body,
        grid=(N // BM,),
        in_specs=[pl.BlockSpec((2, BM, 128), lambda i: (0, i, 0)),
                  pl.BlockSpec((BM, 128), lambda i: (i, 0)),
                  _p_spec(),
                  pl.BlockSpec((128, 256), lambda i: (0, 0)),
                  pl.BlockSpec((1, 256), lambda i: (0, 0))],
        out_specs=pl.BlockSpec((BM, 256), lambda i: (i, 0)),
        out_shape=jax.ShapeDtypeStruct((N, 256), jnp.float32),
    )(s3, zs, p, w3, b3)


def kernel(x, edge_index, W1, b1, W2, b2, W3, b3):
    src = edge_index[0].astype(jnp.int32)
    dst = edge_index[1].astype(jnp.int32)
    pad = E_PAD - E
    fill = jnp.arange(pad, dtype=jnp.int32)
    src_p = jnp.concatenate([src, fill % N])
    dst_p = jnp.concatenate([dst, N + fill % (N_ACC - N)])
    src2 = src_p.reshape(-1, CHUNK)
    srcB = jnp.concatenate([src_p, src_p + N]).reshape(-1, CHUNK)
    dst2 = dst_p.reshape(-1, CHUNK)
    vals = jnp.stack([jnp.ones((CHUNK, 16), jnp.float32),
                      jnp.zeros((CHUNK, 16), jnp.float32)])

    p = _deg(dst2, vals).reshape(2, N, 16)
    y1 = _mm1(x, W1, p)
    s1 = _prop256(y1.reshape(2 * N, 128), srcB, dst2).reshape(2, N, 128)
    y2 = _mm2(s1, y1, p, b1.reshape(1, -1), W2)
    s2 = _prop128p(y2, src2, dst2).reshape(2, N, 128)
    z, zs = _zk(s2, y2, p, b2.reshape(1, -1))
    s3 = _prop128p(zs, src2, dst2).reshape(2, N, 128)
    out = _outk(s3, zs, p, W3, b3.reshape(1, -1))
    return (out, z)
